# baseline (speedup 1.0000x reference)
import jax
import jax.numpy as jnp
from jax import lax
from jax.experimental import pallas as pl
from jax.experimental.pallas import tpu as pltpu
from jax.experimental.pallas import tpu_sc as plsc

N_ENT = 50000
N_USERS = 10000
N_ITEMS = 20000
N_REL = 10
DIM = 64
HALF = 32
E_KG = 800000
E_I2U = 600000
EPS = 1e-7

NC = 2
NS = 16
NW = NC * NS
CH = 128

KG_CHG = 196
TILE_KG_G = KG_CHG * CH
EKG_PAD = NW * TILE_KG_G
KG_CHS = 2 * KG_CHG
TILE_KG_S = KG_CHS * CH

I2U_CHG = 147
TILE_I2U_G = I2U_CHG * CH
E2_PAD = NW * TILE_I2U_G
I2U_CHS = 2 * I2U_CHG
TILE_I2U_S = I2U_CHS * CH

ACC_KG = 50048
ZKG = ACC_KG // NS
N_NODE = N_ITEMS + N_USERS
ACC_2 = 30080
Z2 = ACC_2 // NS
CNT_W = 16

EBLK = 2048
N_EBLK = EKG_PAD // EBLK

KI = 7

_SC_MESH = dict(core_axis_name="c", subcore_axis_name="s")


def _sds(shape, dtype=jnp.float32):
    return jax.ShapeDtypeStruct(shape, dtype)


def _norm(x):
    return jnp.sqrt(jnp.sum(x * x, axis=-1, keepdims=True) + 1e-15)


def _proj(x):
    maxnorm = 1.0 - 1e-3
    n = _norm(x)
    return jnp.where(n > maxnorm, x / n * maxnorm, x)


def _l2n(x):
    return x / _norm(x)


def _sc_gather_kg(P, ego, srcg4, dstg4):
    def body(p_h, ego_h, src_h, dst_h, ps_out, xd_out,
             idx_s, idx_d, rows_a, rows_b, sem):
        c = lax.axis_index("c")
        s = lax.axis_index("s")
        wid = s * NC + c
        base = wid * TILE_KG_G

        def outer(o, carry):
            pltpu.sync_copy(src_h.at[wid, o], idx_s)
            pltpu.sync_copy(dst_h.at[wid, o], idx_d)
            for kk in range(KI):
                off = base + (o * KI + kk) * CH
                cp_a = pltpu.async_copy(p_h.at[idx_s.at[kk]], rows_a, sem)
                cp_b = pltpu.async_copy(ego_h.at[idx_d.at[kk]], rows_b, sem)
                cp_a.wait()
                pltpu.sync_copy(rows_a, ps_out.at[pl.ds(off, CH)])
                cp_b.wait()
                pltpu.sync_copy(rows_b, xd_out.at[pl.ds(off, CH)])
            return carry

        lax.fori_loop(0, KG_CHG // KI, outer, 0)

    return pl.kernel(
        body,
        out_type=(_sds((EKG_PAD, DIM)), _sds((EKG_PAD, DIM))),
        mesh=plsc.VectorSubcoreMesh(**_SC_MESH),
        compiler_params=pltpu.CompilerParams(use_tc_tiling_on_sc=False),
        scratch_types=[
            pltpu.VMEM((KI, CH), jnp.int32),
            pltpu.VMEM((KI, CH), jnp.int32),
            pltpu.VMEM((CH, DIM), jnp.float32),
            pltpu.VMEM((CH, DIM), jnp.float32),
            pltpu.SemaphoreType.DMA,
        ],
    )(P, ego, srcg4, dstg4)


def _sc_scatter_kg(ts2, srcs4, zeros_kg):
    def body(ts_h, src_h, z_h, out_h, idx_v, rows_v, acc):
        c = lax.axis_index("c")
        s = lax.axis_index("s")
        pltpu.sync_copy(z_h.at[pl.ds(s * ZKG, ZKG)],
                        acc.at[pl.ds(s * ZKG, ZKG)])
        plsc.subcore_barrier()
        base = s * TILE_KG_S

        def outer(o, carry):
            pltpu.sync_copy(src_h.at[s, o], idx_v)
            for kk in range(KI):
                off = base + (o * KI + kk) * CH
                pltpu.sync_copy(ts_h.at[c, pl.ds(off, CH)], rows_v)
                pltpu.sync_copy(rows_v, acc.at[idx_v.at[kk]], add=True)
            return carry

        lax.fori_loop(0, KG_CHS // KI, outer, 0)
        plsc.subcore_barrier()
        pltpu.sync_copy(acc.at[pl.ds(s * ZKG, ZKG)],
                        out_h.at[c, pl.ds(s * ZKG, ZKG)])

    return pl.kernel(
        body,
        out_type=_sds((NC, ACC_KG, HALF)),
        mesh=plsc.VectorSubcoreMesh(**_SC_MESH),
        compiler_params=pltpu.CompilerParams(use_tc_tiling_on_sc=False),
        scratch_types=[
            pltpu.VMEM((KI, CH), jnp.int32),
            pltpu.VMEM((CH, HALF), jnp.float32),
            pltpu.VMEM_SHARED((ACC_KG, HALF), jnp.float32),
        ],
    )(ts2, srcs4, zeros_kg)


def _sc_i2u(node2, src5, dst4, zeros_2):
    def body(n_h, src_h, dst_h, z_h, out_h, idx_s, idx_d, rows_v, acc, sem):
        c = lax.axis_index("c")
        s = lax.axis_index("s")
        pltpu.sync_copy(z_h.at[pl.ds(s * Z2, Z2)], acc.at[pl.ds(s * Z2, Z2)])
        plsc.subcore_barrier()

        def outer(o, carry):
            pltpu.sync_copy(src_h.at[c, s, o], idx_s)
            pltpu.sync_copy(dst_h.at[s, o], idx_d)
            for kk in range(KI):
                pltpu.async_copy(n_h.at[idx_s.at[kk]], rows_v, sem).wait()
                pltpu.sync_copy(rows_v, acc.at[idx_d.at[kk]], add=True)
            return carry

        lax.fori_loop(0, I2U_CHS // KI, outer, 0)
        plsc.subcore_barrier()
        pltpu.sync_copy(acc.at[pl.ds(s * Z2, Z2)],
                        out_h.at[c, pl.ds(s * Z2, Z2)])

    return pl.kernel(
        body,
        out_type=_sds((NC, ACC_2, HALF)),
        mesh=plsc.VectorSubcoreMesh(**_SC_MESH),
        compiler_params=pltpu.CompilerParams(use_tc_tiling_on_sc=False),
        scratch_types=[
            pltpu.VMEM((KI, CH), jnp.int32),
            pltpu.VMEM((KI, CH), jnp.int32),
            pltpu.VMEM((CH, HALF), jnp.float32),
            pltpu.VMEM_SHARED((ACC_2, HALF), jnp.float32),
            pltpu.SemaphoreType.DMA,
        ],
    )(node2, src5, dst4, zeros_2)


def _sc_counts(srccnt4, dstcnt4, ones_h, zkg16, z2_16):
    def body(src_h, dst_h, on_h, zk_h, z2_h, okg, o2,
             idx_kg, idx_2, ones_v, acck, acc2):
        c = lax.axis_index("c")
        s = lax.axis_index("s")
        wid = s * NC + c
        pltpu.sync_copy(zk_h.at[pl.ds(s * ZKG, ZKG)],
                        acck.at[pl.ds(s * ZKG, ZKG)])
        pltpu.sync_copy(z2_h.at[pl.ds(s * Z2, Z2)],
                        acc2.at[pl.ds(s * Z2, Z2)])
        pltpu.sync_copy(on_h, ones_v)
        plsc.subcore_barrier()

        def chunk_kg(o, carry):
            pltpu.sync_copy(src_h.at[wid, o], idx_kg)
            for kk in range(KI):
                pltpu.sync_copy(ones_v, acck.at[idx_kg.at[kk]], add=True)
            return carry

        lax.fori_loop(0, KG_CHG // KI, chunk_kg, 0)

        def chunk_2(o, carry):
            pltpu.sync_copy(dst_h.at[wid, o], idx_2)
            for kk in range(KI):
                pltpu.sync_copy(ones_v, acc2.at[idx_2.at[kk]], add=True)
            return carry

        lax.fori_loop(0, I2U_CHG // KI, chunk_2, 0)
        plsc.subcore_barrier()
        pltpu.sync_copy(acck.at[pl.ds(s * ZKG, ZKG)],
                        okg.at[c, pl.ds(s * ZKG, ZKG)])
        pltpu.sync_copy(acc2.at[pl.ds(s * Z2, Z2)],
                        o2.at[c, pl.ds(s * Z2, Z2)])

    return pl.kernel(
        body,
        out_type=(_sds((NC, ACC_KG, CNT_W)), _sds((NC, ACC_2, CNT_W))),
        mesh=plsc.VectorSubcoreMesh(**_SC_MESH),
        compiler_params=pltpu.CompilerParams(use_tc_tiling_on_sc=False),
        scratch_types=[
            pltpu.VMEM((KI, CH), jnp.int32),
            pltpu.VMEM((KI, CH), jnp.int32),
            pltpu.VMEM((CH, CNT_W), jnp.float32),
            pltpu.VMEM_SHARED((ACC_KG, CNT_W), jnp.float32),
            pltpu.VMEM_SHARED((ACC_2, CNT_W), jnp.float32),
        ],
    )(srccnt4, dstcnt4, ones_h, zkg16, z2_16)


def _tc_expmap0(ego):
    def body(x_ref, o_ref):
        x = x_ref[...]
        n = _norm(x)
        o_ref[...] = _proj(jnp.tanh(n) * x / n)

    return pl.pallas_call(
        body,
        grid=(25,),
        in_specs=[pl.BlockSpec((2000, DIM), lambda i: (i, 0))],
        out_specs=pl.BlockSpec((2000, DIM), lambda i: (i, 0)),
        out_shape=_sds((N_ENT, DIM)),
    )(ego)


def _tc_relsel(relm, rtx):
    def body(rm_ref, rt_ref, o_ref):
        rm = rm_ref[...]
        oh = (lax.broadcasted_iota(jnp.int32, (16, EBLK), 0)
              == (rm + 2)).astype(jnp.float32)
        o_ref[...] = jax.lax.dot_general(
            rt_ref[...], oh, (((1,), (0,)), ((), ())),
            preferred_element_type=jnp.float32,
            precision=jax.lax.Precision.HIGHEST)

    return pl.pallas_call(
        body,
        grid=(N_EBLK,),
        in_specs=[
            pl.BlockSpec((1, EBLK), lambda i: (0, i)),
            pl.BlockSpec((72, 16), lambda i: (0, 0)),
        ],
        out_specs=pl.BlockSpec((72, EBLK), lambda i: (0, i)),
        out_shape=_sds((72, EKG_PAD)),
    )(relm, rtx)


def _tc_edge_math(p_src, x_dst, relx):
    def body(p_ref, x_ref, r_ref, rn_ref, o_ref):
        eye = (lax.broadcasted_iota(jnp.int32, (DIM, DIM), 0)
               == lax.broadcasted_iota(jnp.int32, (DIM, DIM), 1)
               ).astype(jnp.float32)
        dnt = (((1,), (1,)), ((), ()))

        def tr_in(v):
            return lax.dot_general(eye, v, dnt,
                                   preferred_element_type=jnp.float32,
                                   precision=jax.lax.Precision.HIGHEST)

        p = tr_in(p_ref[...])
        xd = tr_in(x_ref[...])
        rl = r_ref[...]
        mx = 1.0 - 1e-3

        def rsum(v):
            return jnp.sum(v, axis=0, keepdims=True)

        P2 = rsum(p * p)
        X2 = rsum(xd * xd)
        R2 = rn_ref[0:1, :]
        PX = rsum(p * xd)
        PR = rsum(p * rl)
        XR = rsum(xd * rl)

        lam = 2.0 / jnp.maximum(1.0 - P2, 1e-10)

        def expm_coeffs(U2, PU):
            n_u = jnp.sqrt(U2 + 1e-15)
            t = jnp.tanh(lam * n_u / 2.0)
            sc = t / n_u
            s2 = sc * sc * U2
            dps = sc * PU
            a = 1.0 + 2.0 * dps + s2
            b = 1.0 - P2
            iden = 1.0 / jnp.maximum(1.0 + 2.0 * dps + P2 * s2, 1e-10)
            s_m = jnp.maximum(
                iden * iden * (a * a * P2 + 2.0 * a * b * dps + b * b * s2),
                0.0)
            n_m = jnp.sqrt(s_m + 1e-15)
            f = jnp.where(n_m > mx, mx / n_m, 1.0)
            return f * iden * a, f * iden * b * sc, f * f * s_m

        Ap, Ax, SA = expm_coeffs(X2, PX)
        Bp, Br, SB = expm_coeffs(R2, PR)

        dab = (Ap * Bp * P2 + Ap * Br * PR + Ax * Bp * PX + Ax * Br * XR)
        a1 = 1.0 + 2.0 * dab + SB
        b1 = 1.0 - SA
        id1 = 1.0 / jnp.maximum(1.0 + 2.0 * dab + SA * SB, 1e-10)
        s_m1 = jnp.maximum(
            id1 * id1 * (a1 * a1 * SA + 2.0 * a1 * b1 * dab + b1 * b1 * SB),
            0.0)
        n_m1 = jnp.sqrt(s_m1 + 1e-15)
        f1 = jnp.where(n_m1 > mx, mx / n_m1, 1.0)
        g1 = f1 * id1
        Yp = g1 * (a1 * Ap + b1 * Bp)
        Yx = g1 * a1 * Ax
        Yr = g1 * b1 * Br
        SY = f1 * f1 * s_m1

        dpy = Yp * P2 + Yx * PX + Yr * PR
        a2 = 1.0 - 2.0 * dpy + SY
        b2 = 1.0 - P2
        id2 = 1.0 / jnp.maximum(1.0 - 2.0 * dpy + P2 * SY, 1e-10)
        s_sub = jnp.maximum(
            id2 * id2 * (a2 * a2 * P2 - 2.0 * a2 * b2 * dpy + b2 * b2 * SY),
            0.0)
        n_sub = jnp.sqrt(s_sub + 1e-15)

        cn = jnp.clip(n_sub, -1.0 + EPS, 1.0 - EPS)
        at = 0.5 * jnp.log((1.0 + cn) / (1.0 - cn))
        k = jnp.maximum(1.0 - P2, 1e-10) * at / n_sub
        kid2 = k * id2
        Cp = kid2 * (b2 * Yp - a2)
        Cx = kid2 * b2 * Yx
        Cr = kid2 * b2 * Yr

        tsT = Cp * p + Cx * xd + Cr * rl
        ts = lax.dot_general(tsT, eye, (((0,), (0,)), ((), ())),
                             preferred_element_type=jnp.float32,
                             precision=jax.lax.Precision.HIGHEST)
        o_ref[0] = ts[:, :HALF]
        o_ref[1] = ts[:, HALF:]

    return pl.pallas_call(
        body,
        grid=(N_EBLK,),
        in_specs=[
            pl.BlockSpec((EBLK, DIM), lambda i: (i, 0)),
            pl.BlockSpec((EBLK, DIM), lambda i: (i, 0)),
            pl.BlockSpec((DIM, EBLK), lambda i: (0, i)),
            pl.BlockSpec((8, EBLK), lambda i: (8, i)),
        ],
        out_specs=pl.BlockSpec((NC, EBLK, HALF), lambda i: (0, i, 0)),
        out_shape=_sds((NC, EKG_PAD, HALF)),
    )(p_src, x_dst, relx, relx)


def _tc_node(ego_items, item_cf, u_embed, g1, g2):
    nb_i = N_ITEMS // 1000
    nb = N_NODE // 1000

    def body(e_ref, cf_ref, u_ref, g1_ref, g2_ref, lo_ref, hi_ref):
        i = pl.program_id(0)

        @pl.when(i < nb_i)
        def _():
            e = e_ref[...]
            icf = cf_ref[...]
            dn = (((1,), (1,)), ((), ()))
            z = (lax.dot_general(e, g1_ref[...], dn,
                                 preferred_element_type=jnp.float32,
                                 precision=jax.lax.Precision.HIGHEST)
                 + lax.dot_general(icf, g2_ref[...], dn,
                                   preferred_element_type=jnp.float32,
                                   precision=jax.lax.Precision.HIGHEST))
            gi = jax.nn.sigmoid(z)
            fus = gi * e + (1.0 - gi) * icf
            lo_ref[...] = fus[:, :HALF]
            hi_ref[...] = fus[:, HALF:]

        @pl.when(i >= nb_i)
        def _():
            u = u_ref[...]
            lo_ref[...] = u[:, :HALF]
            hi_ref[...] = u[:, HALF:]

    return pl.pallas_call(
        body,
        grid=(nb,),
        in_specs=[
            pl.BlockSpec((1000, DIM), lambda i: (jnp.minimum(i, nb_i - 1), 0)),
            pl.BlockSpec((1000, DIM), lambda i: (jnp.minimum(i, nb_i - 1), 0)),
            pl.BlockSpec((1000, DIM), lambda i: (jnp.maximum(i - nb_i, 0), 0)),
            pl.BlockSpec((DIM, DIM), lambda i: (0, 0)),
            pl.BlockSpec((DIM, DIM), lambda i: (0, 0)),
        ],
        out_specs=[
            pl.BlockSpec((1000, HALF), lambda i: (i, 0)),
            pl.BlockSpec((1000, HALF), lambda i: (i, 0)),
        ],
        out_shape=(_sds((N_NODE, HALF)), _sds((N_NODE, HALF))),
    )(ego_items, item_cf, u_embed, g1, g2)


def _finalize_body(s2_ref, c2_ref, res_ref, new_ref, ro_ref):
    s = jnp.concatenate([s2_ref[0], s2_ref[1]], axis=1)
    cnt = c2_ref[0][:, :1] + c2_ref[1][:, :1]
    m = s / jnp.maximum(cnt, 1.0)
    e = _l2n(m)
    new_ref[...] = e
    ro_ref[...] = res_ref[...] + e


def _tc_fin(s2, c2, res, row_off, n_rows):
    nb = n_rows // 1000

    return pl.pallas_call(
        _finalize_body,
        grid=(nb,),
        in_specs=[
            pl.BlockSpec((NC, 1000, HALF), lambda i: (0, i + row_off, 0)),
            pl.BlockSpec((NC, 1000, CNT_W), lambda i: (0, i + row_off, 0)),
            pl.BlockSpec((1000, DIM), lambda i: (i, 0)),
        ],
        out_specs=[
            pl.BlockSpec((1000, DIM), lambda i: (i, 0)),
            pl.BlockSpec((1000, DIM), lambda i: (i, 0)),
        ],
        out_shape=(_sds((n_rows, DIM)), _sds((n_rows, DIM))),
    )(s2, c2, res)


def kernel(kg_edge_index, kg_edge_type, i2u_edge_index, entity_user_embed,
           relation_embed, items_embed_cf, gate1_w, gate2_w):
    src = kg_edge_index[0].astype(jnp.int32)
    dst = kg_edge_index[1].astype(jnp.int32)
    i2s = i2u_edge_index[0].astype(jnp.int32)
    i2d = i2u_edge_index[1].astype(jnp.int32)
    ktype = kg_edge_type.astype(jnp.int32)

    pad_kg = EKG_PAD - E_KG
    pad_2 = E2_PAD - E_I2U
    srcg4 = jnp.pad(src, (0, pad_kg)).reshape(NW, KG_CHG // KI, KI, CH)
    dstg4 = jnp.pad(dst, (0, pad_kg)).reshape(NW, KG_CHG // KI, KI, CH)
    src_s = jnp.pad(src, (0, pad_kg), constant_values=N_ENT)
    srcs4 = src_s.reshape(NS, KG_CHS // KI, KI, CH)
    srccnt4 = src_s.reshape(NW, KG_CHG // KI, KI, CH)
    s2g = jnp.pad(i2s, (0, pad_2))
    src5 = jnp.stack([s2g, s2g + N_NODE]).reshape(
        NC, NS, I2U_CHS // KI, KI, CH)
    d2_s = jnp.pad(i2d, (0, pad_2), constant_values=N_NODE)
    dst4 = d2_s.reshape(NS, I2U_CHS // KI, KI, CH)
    dstcnt4 = d2_s.reshape(NW, I2U_CHG // KI, KI, CH)
    relm = jnp.pad(ktype, (0, pad_kg)).reshape(1, EKG_PAD)
    rtx = jnp.zeros((72, 16), jnp.float32)
    rtx = rtx.at[:DIM, :N_REL].set(relation_embed.T)
    rtx = rtx.at[DIM, :N_REL].set(jnp.sum(relation_embed ** 2, axis=1))

    zeros_kg = jnp.zeros((ACC_KG, HALF), jnp.float32)
    zeros_2 = jnp.zeros((ACC_2, HALF), jnp.float32)
    zkg16 = jnp.zeros((ACC_KG, CNT_W), jnp.float32)
    z2_16 = jnp.zeros((ACC_2, CNT_W), jnp.float32)
    ones16 = jnp.ones((CH, CNT_W), jnp.float32)

    ckg, c2 = _sc_counts(srccnt4, dstcnt4, ones16, zkg16, z2_16)
    relx = _tc_relsel(relm, rtx)

    ego = entity_user_embed[:N_ENT]
    u_embed = entity_user_embed[N_ENT:]
    item_cf = items_embed_cf
    ent_res, user_res, item_res = ego, u_embed, item_cf

    for layer in range(2):
        P = _tc_expmap0(ego)
        p_src, x_dst = _sc_gather_kg(P, ego, srcg4, dstg4)
        ts2 = _tc_edge_math(p_src, x_dst, relx)
        kg2 = _sc_scatter_kg(ts2, srcs4, zeros_kg)
        n_lo, n_hi = _tc_node(ego[:N_ITEMS], item_cf, u_embed,
                              gate1_w[layer], gate2_w[layer])
        node2 = jnp.concatenate([n_lo, n_hi], axis=0)
        a2 = _sc_i2u(node2, src5, dst4, zeros_2)
        ego, ent_res = _tc_fin(kg2, ckg, ent_res, 0, N_ENT)
        item_cf, item_res = _tc_fin(a2, c2, item_res, 0, N_ITEMS)
        u_embed, user_res = _tc_fin(a2, c2, user_res, N_ITEMS // 1000,
                                    N_USERS)

    return ent_res, user_res, item_res


# speedup vs baseline: 1.0007x; 1.0007x over previous
import jax
import jax.numpy as jnp
from jax import lax
from jax.experimental import pallas as pl
from jax.experimental.pallas import tpu as pltpu
from jax.experimental.pallas import tpu_sc as plsc

N_ENT = 50000
N_USERS = 10000
N_ITEMS = 20000
N_REL = 10
DIM = 64
HALF = 32
E_KG = 800000
E_I2U = 600000
EPS = 1e-7

NC = 2
NS = 16
NW = NC * NS
CH = 128

KG_CHG = 196
TILE_KG_G = KG_CHG * CH
EKG_PAD = NW * TILE_KG_G
KG_CHS = 2 * KG_CHG
TILE_KG_S = KG_CHS * CH

I2U_CHG = 147
TILE_I2U_G = I2U_CHG * CH
E2_PAD = NW * TILE_I2U_G
I2U_CHS = 2 * I2U_CHG
TILE_I2U_S = I2U_CHS * CH

ACC_KG = 50048
ZKG = ACC_KG // NS
N_NODE = N_ITEMS + N_USERS
ACC_2 = 30080
Z2 = ACC_2 // NS
CNT_W = 16

EBLK = 2048
N_EBLK = EKG_PAD // EBLK

KI = 7

_SC_MESH = dict(core_axis_name="c", subcore_axis_name="s")


def _sds(shape, dtype=jnp.float32):
    return jax.ShapeDtypeStruct(shape, dtype)


def _norm(x):
    return jnp.sqrt(jnp.sum(x * x, axis=-1, keepdims=True) + 1e-15)


def _proj(x):
    maxnorm = 1.0 - 1e-3
    n = _norm(x)
    return jnp.where(n > maxnorm, x / n * maxnorm, x)


def _l2n(x):
    return x / _norm(x)


def _sc_gather_kg(P, ego, srcg4, dstg4):
    def body(p_h, ego_h, src_h, dst_h, ps_out, xd_out,
             idx_s, idx_d, rows_a, rows_b, sem):
        c = lax.axis_index("c")
        s = lax.axis_index("s")
        wid = s * NC + c
        base = wid * TILE_KG_G

        def outer(o, carry):
            pltpu.sync_copy(src_h.at[wid, o], idx_s)
            pltpu.sync_copy(dst_h.at[wid, o], idx_d)
            for kk in range(KI):
                off = base + (o * KI + kk) * CH
                cp_a = pltpu.async_copy(p_h.at[idx_s.at[kk]], rows_a, sem)
                cp_b = pltpu.async_copy(ego_h.at[idx_d.at[kk]], rows_b, sem)
                cp_a.wait()
                pltpu.sync_copy(rows_a, ps_out.at[pl.ds(off, CH)])
                cp_b.wait()
                pltpu.sync_copy(rows_b, xd_out.at[pl.ds(off, CH)])
            return carry

        lax.fori_loop(0, KG_CHG // KI, outer, 0)

    return pl.kernel(
        body,
        out_type=(_sds((EKG_PAD, DIM)), _sds((EKG_PAD, DIM))),
        mesh=plsc.VectorSubcoreMesh(**_SC_MESH),
        compiler_params=pltpu.CompilerParams(use_tc_tiling_on_sc=False),
        scratch_types=[
            pltpu.VMEM((KI, CH), jnp.int32),
            pltpu.VMEM((KI, CH), jnp.int32),
            pltpu.VMEM((CH, DIM), jnp.float32),
            pltpu.VMEM((CH, DIM), jnp.float32),
            pltpu.SemaphoreType.DMA,
        ],
    )(P, ego, srcg4, dstg4)


def _sc_scatter_kg(ts2, srcs4, zeros_kg):
    def body(ts_h, src_h, z_h, out_h, idx_v, rows_v, acc):
        c = lax.axis_index("c")
        s = lax.axis_index("s")
        pltpu.sync_copy(z_h.at[pl.ds(s * ZKG, ZKG)],
                        acc.at[pl.ds(s * ZKG, ZKG)])
        plsc.subcore_barrier()
        base = s * TILE_KG_S

        def outer(o, carry):
            pltpu.sync_copy(src_h.at[s, o], idx_v)
            for kk in range(KI):
                off = base + (o * KI + kk) * CH
                pltpu.sync_copy(ts_h.at[c, pl.ds(off, CH)], rows_v)
                pltpu.sync_copy(rows_v, acc.at[idx_v.at[kk]], add=True)
            return carry

        lax.fori_loop(0, KG_CHS // KI, outer, 0)
        plsc.subcore_barrier()
        pltpu.sync_copy(acc.at[pl.ds(s * ZKG, ZKG)],
                        out_h.at[c, pl.ds(s * ZKG, ZKG)])

    return pl.kernel(
        body,
        out_type=_sds((NC, ACC_KG, HALF)),
        mesh=plsc.VectorSubcoreMesh(**_SC_MESH),
        compiler_params=pltpu.CompilerParams(use_tc_tiling_on_sc=False),
        scratch_types=[
            pltpu.VMEM((KI, CH), jnp.int32),
            pltpu.VMEM((CH, HALF), jnp.float32),
            pltpu.VMEM_SHARED((ACC_KG, HALF), jnp.float32),
        ],
    )(ts2, srcs4, zeros_kg)


def _sc_i2u(node2, src5, dst4, zeros_2):
    def body(n_h, src_h, dst_h, z_h, out_h, idx_s, idx_d, rows_v, acc, sem):
        c = lax.axis_index("c")
        s = lax.axis_index("s")
        pltpu.sync_copy(z_h.at[pl.ds(s * Z2, Z2)], acc.at[pl.ds(s * Z2, Z2)])
        plsc.subcore_barrier()

        def outer(o, carry):
            pltpu.sync_copy(src_h.at[c, s, o], idx_s)
            pltpu.sync_copy(dst_h.at[s, o], idx_d)
            for kk in range(KI):
                pltpu.async_copy(n_h.at[idx_s.at[kk]], rows_v, sem).wait()
                pltpu.sync_copy(rows_v, acc.at[idx_d.at[kk]], add=True)
            return carry

        lax.fori_loop(0, I2U_CHS // KI, outer, 0)
        plsc.subcore_barrier()
        pltpu.sync_copy(acc.at[pl.ds(s * Z2, Z2)],
                        out_h.at[c, pl.ds(s * Z2, Z2)])

    return pl.kernel(
        body,
        out_type=_sds((NC, ACC_2, HALF)),
        mesh=plsc.VectorSubcoreMesh(**_SC_MESH),
        compiler_params=pltpu.CompilerParams(use_tc_tiling_on_sc=False),
        scratch_types=[
            pltpu.VMEM((KI, CH), jnp.int32),
            pltpu.VMEM((KI, CH), jnp.int32),
            pltpu.VMEM((CH, HALF), jnp.float32),
            pltpu.VMEM_SHARED((ACC_2, HALF), jnp.float32),
            pltpu.SemaphoreType.DMA,
        ],
    )(node2, src5, dst4, zeros_2)


def _sc_counts(srccnt4, dstcnt4, ones_h, zkg16, z2_16):
    def body(src_h, dst_h, on_h, zk_h, z2_h, okg, o2,
             idx_kg, idx_2, ones_v, acck, acc2):
        c = lax.axis_index("c")
        s = lax.axis_index("s")
        wid = s * NC + c
        pltpu.sync_copy(zk_h.at[pl.ds(s * ZKG, ZKG)],
                        acck.at[pl.ds(s * ZKG, ZKG)])
        pltpu.sync_copy(z2_h.at[pl.ds(s * Z2, Z2)],
                        acc2.at[pl.ds(s * Z2, Z2)])
        pltpu.sync_copy(on_h, ones_v)
        plsc.subcore_barrier()

        def chunk_kg(o, carry):
            pltpu.sync_copy(src_h.at[wid, o], idx_kg)
            for kk in range(KI):
                pltpu.sync_copy(ones_v, acck.at[idx_kg.at[kk]], add=True)
            return carry

        lax.fori_loop(0, KG_CHG // KI, chunk_kg, 0)

        def chunk_2(o, carry):
            pltpu.sync_copy(dst_h.at[wid, o], idx_2)
            for kk in range(KI):
                pltpu.sync_copy(ones_v, acc2.at[idx_2.at[kk]], add=True)
            return carry

        lax.fori_loop(0, I2U_CHG // KI, chunk_2, 0)
        plsc.subcore_barrier()
        pltpu.sync_copy(acck.at[pl.ds(s * ZKG, ZKG)],
                        okg.at[c, pl.ds(s * ZKG, ZKG)])
        pltpu.sync_copy(acc2.at[pl.ds(s * Z2, Z2)],
                        o2.at[c, pl.ds(s * Z2, Z2)])

    return pl.kernel(
        body,
        out_type=(_sds((NC, ACC_KG, CNT_W)), _sds((NC, ACC_2, CNT_W))),
        mesh=plsc.VectorSubcoreMesh(**_SC_MESH),
        compiler_params=pltpu.CompilerParams(use_tc_tiling_on_sc=False),
        scratch_types=[
            pltpu.VMEM((KI, CH), jnp.int32),
            pltpu.VMEM((KI, CH), jnp.int32),
            pltpu.VMEM((CH, CNT_W), jnp.float32),
            pltpu.VMEM_SHARED((ACC_KG, CNT_W), jnp.float32),
            pltpu.VMEM_SHARED((ACC_2, CNT_W), jnp.float32),
        ],
    )(srccnt4, dstcnt4, ones_h, zkg16, z2_16)


def _tc_expmap0(ego):
    def body(x_ref, o_ref):
        x = x_ref[...]
        n = _norm(x)
        o_ref[...] = _proj(jnp.tanh(n) * x / n)

    return pl.pallas_call(
        body,
        grid=(25,),
        in_specs=[pl.BlockSpec((2000, DIM), lambda i: (i, 0))],
        out_specs=pl.BlockSpec((2000, DIM), lambda i: (i, 0)),
        out_shape=_sds((N_ENT, DIM)),
    )(ego)


def _tc_relsel(relm, rtx):
    def body(rm_ref, rt_ref, o_ref):
        rm = rm_ref[...]
        oh = (lax.broadcasted_iota(jnp.int32, (16, EBLK), 0)
              == (rm + 2)).astype(jnp.float32)
        o_ref[...] = jax.lax.dot_general(
            rt_ref[...], oh, (((1,), (0,)), ((), ())),
            preferred_element_type=jnp.float32,
            precision=jax.lax.Precision.HIGHEST)

    return pl.pallas_call(
        body,
        grid=(N_EBLK,),
        in_specs=[
            pl.BlockSpec((1, EBLK), lambda i: (0, i)),
            pl.BlockSpec((72, 16), lambda i: (0, 0)),
        ],
        out_specs=pl.BlockSpec((72, EBLK), lambda i: (0, i)),
        out_shape=_sds((72, EKG_PAD)),
    )(relm, rtx)


def _tc_edge_math(p_src, x_dst, relx):
    def body(p_ref, x_ref, r_ref, rn_ref, o_ref):
        eye = (lax.broadcasted_iota(jnp.int32, (DIM, DIM), 0)
               == lax.broadcasted_iota(jnp.int32, (DIM, DIM), 1)
               ).astype(jnp.float32)
        dnt = (((1,), (1,)), ((), ()))

        def tr_in(v):
            return lax.dot_general(eye, v, dnt,
                                   preferred_element_type=jnp.float32,
                                   precision=jax.lax.Precision.HIGHEST)

        p = tr_in(p_ref[...])
        xd = tr_in(x_ref[...])
        rl = r_ref[...]
        mx = 1.0 - 1e-3

        def rsum(v):
            return jnp.sum(v, axis=0, keepdims=True)

        P2 = rsum(p * p)
        X2 = rsum(xd * xd)
        R2 = rn_ref[0:1, :]
        PX = rsum(p * xd)
        PR = rsum(p * rl)
        XR = rsum(xd * rl)

        lam = 2.0 / jnp.maximum(1.0 - P2, 1e-10)

        def expm_coeffs(U2, PU):
            n_u = jnp.sqrt(U2 + 1e-15)
            t = jnp.tanh(lam * n_u / 2.0)
            sc = t / n_u
            s2 = sc * sc * U2
            dps = sc * PU
            a = 1.0 + 2.0 * dps + s2
            b = 1.0 - P2
            iden = 1.0 / jnp.maximum(1.0 + 2.0 * dps + P2 * s2, 1e-10)
            s_m = jnp.maximum(
                iden * iden * (a * a * P2 + 2.0 * a * b * dps + b * b * s2),
                0.0)
            n_m = jnp.sqrt(s_m + 1e-15)
            f = jnp.where(n_m > mx, mx / n_m, 1.0)
            return f * iden * a, f * iden * b * sc, f * f * s_m

        Ap, Ax, SA = expm_coeffs(X2, PX)
        Bp, Br, SB = expm_coeffs(R2, PR)

        dab = (Ap * Bp * P2 + Ap * Br * PR + Ax * Bp * PX + Ax * Br * XR)
        a1 = 1.0 + 2.0 * dab + SB
        b1 = 1.0 - SA
        id1 = 1.0 / jnp.maximum(1.0 + 2.0 * dab + SA * SB, 1e-10)
        s_m1 = jnp.maximum(
            id1 * id1 * (a1 * a1 * SA + 2.0 * a1 * b1 * dab + b1 * b1 * SB),
            0.0)
        n_m1 = jnp.sqrt(s_m1 + 1e-15)
        f1 = jnp.where(n_m1 > mx, mx / n_m1, 1.0)
        g1 = f1 * id1
        Yp = g1 * (a1 * Ap + b1 * Bp)
        Yx = g1 * a1 * Ax
        Yr = g1 * b1 * Br
        SY = f1 * f1 * s_m1

        dpy = Yp * P2 + Yx * PX + Yr * PR
        a2 = 1.0 - 2.0 * dpy + SY
        b2 = 1.0 - P2
        id2 = 1.0 / jnp.maximum(1.0 - 2.0 * dpy + P2 * SY, 1e-10)
        s_sub = jnp.maximum(
            id2 * id2 * (a2 * a2 * P2 - 2.0 * a2 * b2 * dpy + b2 * b2 * SY),
            0.0)
        n_sub = jnp.sqrt(s_sub + 1e-15)

        cn = jnp.clip(n_sub, -1.0 + EPS, 1.0 - EPS)
        at = 0.5 * jnp.log((1.0 + cn) / (1.0 - cn))
        k = jnp.maximum(1.0 - P2, 1e-10) * at / n_sub
        kid2 = k * id2
        Cp = kid2 * (b2 * Yp - a2)
        Cx = kid2 * b2 * Yx
        Cr = kid2 * b2 * Yr

        tsT = Cp * p + Cx * xd + Cr * rl
        ts = lax.dot_general(tsT, eye, (((0,), (0,)), ((), ())),
                             preferred_element_type=jnp.float32,
                             precision=jax.lax.Precision.HIGHEST)
        o_ref[0] = ts[:, :HALF]
        o_ref[1] = ts[:, HALF:]

    return pl.pallas_call(
        body,
        grid=(N_EBLK,),
        in_specs=[
            pl.BlockSpec((EBLK, DIM), lambda i: (i, 0)),
            pl.BlockSpec((EBLK, DIM), lambda i: (i, 0)),
            pl.BlockSpec((DIM, EBLK), lambda i: (0, i)),
            pl.BlockSpec((8, EBLK), lambda i: (8, i)),
        ],
        out_specs=pl.BlockSpec((NC, EBLK, HALF), lambda i: (0, i, 0)),
        out_shape=_sds((NC, EKG_PAD, HALF)),
    )(p_src, x_dst, relx, relx)


def _tc_node(ego_items, item_cf, u_embed, g1, g2):
    nb_i = N_ITEMS // 1000
    nb = N_NODE // 1000

    def body(e_ref, cf_ref, u_ref, g1_ref, g2_ref, lo_ref, hi_ref):
        i = pl.program_id(0)

        @pl.when(i < nb_i)
        def _():
            e = e_ref[...]
            icf = cf_ref[...]
            dn = (((1,), (1,)), ((), ()))
            z = (lax.dot_general(e, g1_ref[...], dn,
                                 preferred_element_type=jnp.float32,
                                 precision=jax.lax.Precision.HIGHEST)
                 + lax.dot_general(icf, g2_ref[...], dn,
                                   preferred_element_type=jnp.float32,
                                   precision=jax.lax.Precision.HIGHEST))
            gi = jax.nn.sigmoid(z)
            fus = gi * e + (1.0 - gi) * icf
            lo_ref[...] = fus[:, :HALF]
            hi_ref[...] = fus[:, HALF:]

        @pl.when(i >= nb_i)
        def _():
            u = u_ref[...]
            lo_ref[...] = u[:, :HALF]
            hi_ref[...] = u[:, HALF:]

    return pl.pallas_call(
        body,
        grid=(nb,),
        in_specs=[
            pl.BlockSpec((1000, DIM), lambda i: (jnp.minimum(i, nb_i - 1), 0)),
            pl.BlockSpec((1000, DIM), lambda i: (jnp.minimum(i, nb_i - 1), 0)),
            pl.BlockSpec((1000, DIM), lambda i: (jnp.maximum(i - nb_i, 0), 0)),
            pl.BlockSpec((DIM, DIM), lambda i: (0, 0)),
            pl.BlockSpec((DIM, DIM), lambda i: (0, 0)),
        ],
        out_specs=[
            pl.BlockSpec((1000, HALF), lambda i: (i, 0)),
            pl.BlockSpec((1000, HALF), lambda i: (i, 0)),
        ],
        out_shape=(_sds((N_NODE, HALF)), _sds((N_NODE, HALF))),
    )(ego_items, item_cf, u_embed, g1, g2)


def _finalize_body(s2_ref, c2_ref, res_ref, new_ref, ro_ref):
    s = jnp.concatenate([s2_ref[0], s2_ref[1]], axis=1)
    cnt = c2_ref[0][:, :1] + c2_ref[1][:, :1]
    m = s / jnp.maximum(cnt, 1.0)
    e = _l2n(m)
    new_ref[...] = e
    ro_ref[...] = res_ref[...] + e


def _tc_fin(s2, c2, res, row_off, n_rows):
    nb = n_rows // 1000

    return pl.pallas_call(
        _finalize_body,
        grid=(nb,),
        in_specs=[
            pl.BlockSpec((NC, 1000, HALF), lambda i: (0, i + row_off, 0)),
            pl.BlockSpec((NC, 1000, CNT_W), lambda i: (0, i + row_off, 0)),
            pl.BlockSpec((1000, DIM), lambda i: (i, 0)),
        ],
        out_specs=[
            pl.BlockSpec((1000, DIM), lambda i: (i, 0)),
            pl.BlockSpec((1000, DIM), lambda i: (i, 0)),
        ],
        out_shape=(_sds((n_rows, DIM)), _sds((n_rows, DIM))),
    )(s2, c2, res)


def kernel(kg_edge_index, kg_edge_type, i2u_edge_index, entity_user_embed,
           relation_embed, items_embed_cf, gate1_w, gate2_w):
    src = kg_edge_index[0].astype(jnp.int32)
    dst = kg_edge_index[1].astype(jnp.int32)
    i2s = i2u_edge_index[0].astype(jnp.int32)
    i2d = i2u_edge_index[1].astype(jnp.int32)
    ktype = kg_edge_type.astype(jnp.int32)

    pad_kg = EKG_PAD - E_KG
    pad_2 = E2_PAD - E_I2U
    srcg4 = jnp.pad(src, (0, pad_kg)).reshape(NW, KG_CHG // KI, KI, CH)
    dstg4 = jnp.pad(dst, (0, pad_kg)).reshape(NW, KG_CHG // KI, KI, CH)
    src_s = jnp.pad(src, (0, pad_kg), constant_values=N_ENT)
    srcs4 = src_s.reshape(NS, KG_CHS // KI, KI, CH)
    srccnt4 = src_s.reshape(NW, KG_CHG // KI, KI, CH)
    s2g = jnp.pad(i2s, (0, pad_2))
    src5 = jnp.stack([s2g, s2g + N_NODE]).reshape(
        NC, NS, I2U_CHS // KI, KI, CH)
    d2_s = jnp.pad(i2d, (0, pad_2), constant_values=N_NODE)
    dst4 = d2_s.reshape(NS, I2U_CHS // KI, KI, CH)
    dstcnt4 = d2_s.reshape(NW, I2U_CHG // KI, KI, CH)
    relm = jnp.pad(ktype, (0, pad_kg)).reshape(1, EKG_PAD)
    rtx = jnp.zeros((72, 16), jnp.float32)
    rtx = rtx.at[:DIM, :N_REL].set(relation_embed.T)
    rtx = rtx.at[DIM, :N_REL].set(jnp.sum(relation_embed ** 2, axis=1))

    zeros_kg = jnp.zeros((ACC_KG, HALF), jnp.float32)
    zeros_2 = jnp.zeros((ACC_2, HALF), jnp.float32)
    zkg16 = jnp.zeros((ACC_KG, CNT_W), jnp.float32)
    z2_16 = jnp.zeros((ACC_2, CNT_W), jnp.float32)
    ones16 = jnp.ones((CH, CNT_W), jnp.float32)

    ckg, c2 = _sc_counts(srccnt4, dstcnt4, ones16, zkg16, z2_16)
    relx = _tc_relsel(relm, rtx)

    ego = entity_user_embed[:N_ENT]
    u_embed = entity_user_embed[N_ENT:]
    item_cf = items_embed_cf
    ent_res, user_res, item_res = ego, u_embed, item_cf

    for layer in range(2):
        P = _tc_expmap0(ego)
        n_lo, n_hi = _tc_node(ego[:N_ITEMS], item_cf, u_embed,
                              gate1_w[layer], gate2_w[layer])
        node2 = jnp.concatenate([n_lo, n_hi], axis=0)
        p_src, x_dst = _sc_gather_kg(P, ego, srcg4, dstg4)
        a2 = _sc_i2u(node2, src5, dst4, zeros_2)
        ts2 = _tc_edge_math(p_src, x_dst, relx)
        kg2 = _sc_scatter_kg(ts2, srcs4, zeros_kg)
        ego, ent_res = _tc_fin(kg2, ckg, ent_res, 0, N_ENT)
        item_cf, item_res = _tc_fin(a2, c2, item_res, 0, N_ITEMS)
        u_embed, user_res = _tc_fin(a2, c2, user_res, N_ITEMS // 1000,
                                    N_USERS)

    return ent_res, user_res, item_res


# speedup vs baseline: 1.1842x; 1.1833x over previous
import jax
import jax.numpy as jnp
from jax import lax
from jax.experimental import pallas as pl
from jax.experimental.pallas import tpu as pltpu
from jax.experimental.pallas import tpu_sc as plsc

N_ENT = 50000
N_USERS = 10000
N_ITEMS = 20000
N_REL = 10
DIM = 64
HALF = 32
E_KG = 800000
E_I2U = 600000
EPS = 1e-7

NC = 2
NS = 16
NW = NC * NS
CH = 128

KG_CHG = 196
TILE_KG_G = KG_CHG * CH
EKG_PAD = NW * TILE_KG_G
KG_CHS = 2 * KG_CHG
TILE_KG_S = KG_CHS * CH

I2U_CHG = 147
TILE_I2U_G = I2U_CHG * CH
E2_PAD = NW * TILE_I2U_G
I2U_CHS = 2 * I2U_CHG
TILE_I2U_S = I2U_CHS * CH

ACC_KG = 50048
ZKG = ACC_KG // NS
N_NODE = N_ITEMS + N_USERS
ACC_2 = 30080
Z2 = ACC_2 // NS
CNT_W = 16

EBLK = 2048
N_EBLK = EKG_PAD // EBLK

KI = 7

_SC_MESH = dict(core_axis_name="c", subcore_axis_name="s")


def _sds(shape, dtype=jnp.float32):
    return jax.ShapeDtypeStruct(shape, dtype)


def _norm(x):
    return jnp.sqrt(jnp.sum(x * x, axis=-1, keepdims=True) + 1e-15)


def _proj(x):
    maxnorm = 1.0 - 1e-3
    n = _norm(x)
    return jnp.where(n > maxnorm, x / n * maxnorm, x)


def _l2n(x):
    return x / _norm(x)


def _sc_gather_kg(P, ego, srcg4, dstg4):
    def body(p_h, ego_h, src_h, dst_h, ps_out, xd_out,
             idx_s, idx_d, pa, ea, pb, eb,
             gsa, gsb, wsa, wsb):
        c = lax.axis_index("c")
        s = lax.axis_index("s")
        wid = s * NC + c
        base = wid * TILE_KG_G
        pltpu.sync_copy(src_h.at[wid], idx_s)
        pltpu.sync_copy(dst_h.at[wid], idx_d)

        def fire_g(j, rp, re, sem):
            pltpu.async_copy(p_h.at[idx_s.at[j]], rp, sem)
            pltpu.async_copy(ego_h.at[idx_d.at[j]], re, sem)

        def wait_g(rp, re, sem):
            pltpu.make_async_copy(p_h.at[pl.ds(0, CH)], rp, sem).wait()
            pltpu.make_async_copy(p_h.at[pl.ds(0, CH)], re, sem).wait()

        def fire_w(j, rp, re, sem):
            off = base + j * CH
            pltpu.async_copy(rp, ps_out.at[pl.ds(off, CH)], sem)
            pltpu.async_copy(re, xd_out.at[pl.ds(off, CH)], sem)

        def wait_w(rp, re, sem):
            pltpu.make_async_copy(rp, ps_out.at[pl.ds(0, CH)], sem).wait()
            pltpu.make_async_copy(re, xd_out.at[pl.ds(0, CH)], sem).wait()

        fire_g(0, pa, ea, gsa)
        fire_g(1, pb, eb, gsb)
        n_it = KG_CHG // 2

        def step(o, carry):
            a = 2 * o
            wait_g(pa, ea, gsa)
            fire_w(a, pa, ea, wsa)
            wait_g(pb, eb, gsb)
            fire_w(a + 1, pb, eb, wsb)
            wait_w(pa, ea, wsa)

            @pl.when(o < n_it - 1)
            def _():
                fire_g(a + 2, pa, ea, gsa)

            wait_w(pb, eb, wsb)

            @pl.when(o < n_it - 1)
            def _():
                fire_g(a + 3, pb, eb, gsb)

            return carry

        lax.fori_loop(0, n_it, step, 0)

    return pl.kernel(
        body,
        out_type=(_sds((EKG_PAD, DIM)), _sds((EKG_PAD, DIM))),
        mesh=plsc.VectorSubcoreMesh(**_SC_MESH),
        compiler_params=pltpu.CompilerParams(use_tc_tiling_on_sc=False),
        scratch_types=[
            pltpu.VMEM((KG_CHG, CH), jnp.int32),
            pltpu.VMEM((KG_CHG, CH), jnp.int32),
            pltpu.VMEM((CH, DIM), jnp.float32),
            pltpu.VMEM((CH, DIM), jnp.float32),
            pltpu.VMEM((CH, DIM), jnp.float32),
            pltpu.VMEM((CH, DIM), jnp.float32),
            pltpu.SemaphoreType.DMA,
            pltpu.SemaphoreType.DMA,
            pltpu.SemaphoreType.DMA,
            pltpu.SemaphoreType.DMA,
        ],
    )(P, ego, srcg4, dstg4)


def _sc_scatter_kg(ts2, srcs4, zeros_kg):
    def body(ts_h, src_h, z_h, out_h, idx_v, rows_v, acc):
        c = lax.axis_index("c")
        s = lax.axis_index("s")
        pltpu.sync_copy(z_h.at[pl.ds(s * ZKG, ZKG)],
                        acc.at[pl.ds(s * ZKG, ZKG)])
        plsc.subcore_barrier()
        base = s * TILE_KG_S

        def outer(o, carry):
            pltpu.sync_copy(src_h.at[s, o], idx_v)
            for kk in range(KI):
                off = base + (o * KI + kk) * CH
                pltpu.sync_copy(ts_h.at[c, pl.ds(off, CH)], rows_v)
                pltpu.sync_copy(rows_v, acc.at[idx_v.at[kk]], add=True)
            return carry

        lax.fori_loop(0, KG_CHS // KI, outer, 0)
        plsc.subcore_barrier()
        pltpu.sync_copy(acc.at[pl.ds(s * ZKG, ZKG)],
                        out_h.at[c, pl.ds(s * ZKG, ZKG)])

    return pl.kernel(
        body,
        out_type=_sds((NC, ACC_KG, HALF)),
        mesh=plsc.VectorSubcoreMesh(**_SC_MESH),
        compiler_params=pltpu.CompilerParams(use_tc_tiling_on_sc=False),
        scratch_types=[
            pltpu.VMEM((KI, CH), jnp.int32),
            pltpu.VMEM((CH, HALF), jnp.float32),
            pltpu.VMEM_SHARED((ACC_KG, HALF), jnp.float32),
        ],
    )(ts2, srcs4, zeros_kg)


def _sc_i2u(node2, src5, dst4, zeros_2):
    def body(n_h, src_h, dst_h, z_h, out_h, idx_s, idx_d, rows_v, acc, sem):
        c = lax.axis_index("c")
        s = lax.axis_index("s")
        pltpu.sync_copy(z_h.at[pl.ds(s * Z2, Z2)], acc.at[pl.ds(s * Z2, Z2)])
        plsc.subcore_barrier()

        def outer(o, carry):
            pltpu.sync_copy(src_h.at[c, s, o], idx_s)
            pltpu.sync_copy(dst_h.at[s, o], idx_d)
            for kk in range(KI):
                pltpu.async_copy(n_h.at[idx_s.at[kk]], rows_v, sem).wait()
                pltpu.sync_copy(rows_v, acc.at[idx_d.at[kk]], add=True)
            return carry

        lax.fori_loop(0, I2U_CHS // KI, outer, 0)
        plsc.subcore_barrier()
        pltpu.sync_copy(acc.at[pl.ds(s * Z2, Z2)],
                        out_h.at[c, pl.ds(s * Z2, Z2)])

    return pl.kernel(
        body,
        out_type=_sds((NC, ACC_2, HALF)),
        mesh=plsc.VectorSubcoreMesh(**_SC_MESH),
        compiler_params=pltpu.CompilerParams(use_tc_tiling_on_sc=False),
        scratch_types=[
            pltpu.VMEM((KI, CH), jnp.int32),
            pltpu.VMEM((KI, CH), jnp.int32),
            pltpu.VMEM((CH, HALF), jnp.float32),
            pltpu.VMEM_SHARED((ACC_2, HALF), jnp.float32),
            pltpu.SemaphoreType.DMA,
        ],
    )(node2, src5, dst4, zeros_2)


def _sc_counts(srccnt4, dstcnt4, ones_h, zkg16, z2_16):
    def body(src_h, dst_h, on_h, zk_h, z2_h, okg, o2,
             idx_kg, idx_2, ones_v, acck, acc2):
        c = lax.axis_index("c")
        s = lax.axis_index("s")
        wid = s * NC + c
        pltpu.sync_copy(zk_h.at[pl.ds(s * ZKG, ZKG)],
                        acck.at[pl.ds(s * ZKG, ZKG)])
        pltpu.sync_copy(z2_h.at[pl.ds(s * Z2, Z2)],
                        acc2.at[pl.ds(s * Z2, Z2)])
        pltpu.sync_copy(on_h, ones_v)
        plsc.subcore_barrier()

        def chunk_kg(o, carry):
            pltpu.sync_copy(src_h.at[wid, o], idx_kg)
            for kk in range(KI):
                pltpu.sync_copy(ones_v, acck.at[idx_kg.at[kk]], add=True)
            return carry

        lax.fori_loop(0, KG_CHG // KI, chunk_kg, 0)

        def chunk_2(o, carry):
            pltpu.sync_copy(dst_h.at[wid, o], idx_2)
            for kk in range(KI):
                pltpu.sync_copy(ones_v, acc2.at[idx_2.at[kk]], add=True)
            return carry

        lax.fori_loop(0, I2U_CHG // KI, chunk_2, 0)
        plsc.subcore_barrier()
        pltpu.sync_copy(acck.at[pl.ds(s * ZKG, ZKG)],
                        okg.at[c, pl.ds(s * ZKG, ZKG)])
        pltpu.sync_copy(acc2.at[pl.ds(s * Z2, Z2)],
                        o2.at[c, pl.ds(s * Z2, Z2)])

    return pl.kernel(
        body,
        out_type=(_sds((NC, ACC_KG, CNT_W)), _sds((NC, ACC_2, CNT_W))),
        mesh=plsc.VectorSubcoreMesh(**_SC_MESH),
        compiler_params=pltpu.CompilerParams(use_tc_tiling_on_sc=False),
        scratch_types=[
            pltpu.VMEM((KI, CH), jnp.int32),
            pltpu.VMEM((KI, CH), jnp.int32),
            pltpu.VMEM((CH, CNT_W), jnp.float32),
            pltpu.VMEM_SHARED((ACC_KG, CNT_W), jnp.float32),
            pltpu.VMEM_SHARED((ACC_2, CNT_W), jnp.float32),
        ],
    )(srccnt4, dstcnt4, ones_h, zkg16, z2_16)


def _tc_expmap0(ego):
    def body(x_ref, o_ref):
        x = x_ref[...]
        n = _norm(x)
        o_ref[...] = _proj(jnp.tanh(n) * x / n)

    return pl.pallas_call(
        body,
        grid=(25,),
        in_specs=[pl.BlockSpec((2000, DIM), lambda i: (i, 0))],
        out_specs=pl.BlockSpec((2000, DIM), lambda i: (i, 0)),
        out_shape=_sds((N_ENT, DIM)),
    )(ego)


def _tc_relsel(relm, rtx):
    def body(rm_ref, rt_ref, o_ref):
        rm = rm_ref[...]
        oh = (lax.broadcasted_iota(jnp.int32, (16, EBLK), 0)
              == (rm + 2)).astype(jnp.float32)
        o_ref[...] = jax.lax.dot_general(
            rt_ref[...], oh, (((1,), (0,)), ((), ())),
            preferred_element_type=jnp.float32,
            precision=jax.lax.Precision.HIGHEST)

    return pl.pallas_call(
        body,
        grid=(N_EBLK,),
        in_specs=[
            pl.BlockSpec((1, EBLK), lambda i: (0, i)),
            pl.BlockSpec((72, 16), lambda i: (0, 0)),
        ],
        out_specs=pl.BlockSpec((72, EBLK), lambda i: (0, i)),
        out_shape=_sds((72, EKG_PAD)),
    )(relm, rtx)


def _tc_edge_math(pT, xdT, relx):
    def body(p_ref, x_ref, r_ref, rn_ref, o_ref):
        p = p_ref[...]
        xd = x_ref[...]
        rl = r_ref[...]
        mx = 1.0 - 1e-3

        def rsum(v):
            return jnp.sum(v, axis=0, keepdims=True)

        P2 = rsum(p * p)
        X2 = rsum(xd * xd)
        R2 = rn_ref[0:1, :]
        PX = rsum(p * xd)
        PR = rsum(p * rl)
        XR = rsum(xd * rl)

        lam = 2.0 / jnp.maximum(1.0 - P2, 1e-10)

        def expm_coeffs(U2, PU):
            n_u = jnp.sqrt(U2 + 1e-15)
            t = jnp.tanh(lam * n_u / 2.0)
            sc = t / n_u
            s2 = sc * sc * U2
            dps = sc * PU
            a = 1.0 + 2.0 * dps + s2
            b = 1.0 - P2
            iden = 1.0 / jnp.maximum(1.0 + 2.0 * dps + P2 * s2, 1e-10)
            s_m = jnp.maximum(
                iden * iden * (a * a * P2 + 2.0 * a * b * dps + b * b * s2),
                0.0)
            n_m = jnp.sqrt(s_m + 1e-15)
            f = jnp.where(n_m > mx, mx / n_m, 1.0)
            return f * iden * a, f * iden * b * sc, f * f * s_m

        Ap, Ax, SA = expm_coeffs(X2, PX)
        Bp, Br, SB = expm_coeffs(R2, PR)

        dab = (Ap * Bp * P2 + Ap * Br * PR + Ax * Bp * PX + Ax * Br * XR)
        a1 = 1.0 + 2.0 * dab + SB
        b1 = 1.0 - SA
        id1 = 1.0 / jnp.maximum(1.0 + 2.0 * dab + SA * SB, 1e-10)
        s_m1 = jnp.maximum(
            id1 * id1 * (a1 * a1 * SA + 2.0 * a1 * b1 * dab + b1 * b1 * SB),
            0.0)
        n_m1 = jnp.sqrt(s_m1 + 1e-15)
        f1 = jnp.where(n_m1 > mx, mx / n_m1, 1.0)
        g1 = f1 * id1
        Yp = g1 * (a1 * Ap + b1 * Bp)
        Yx = g1 * a1 * Ax
        Yr = g1 * b1 * Br
        SY = f1 * f1 * s_m1

        dpy = Yp * P2 + Yx * PX + Yr * PR
        a2 = 1.0 - 2.0 * dpy + SY
        b2 = 1.0 - P2
        id2 = 1.0 / jnp.maximum(1.0 - 2.0 * dpy + P2 * SY, 1e-10)
        s_sub = jnp.maximum(
            id2 * id2 * (a2 * a2 * P2 - 2.0 * a2 * b2 * dpy + b2 * b2 * SY),
            0.0)
        n_sub = jnp.sqrt(s_sub + 1e-15)

        cn = jnp.clip(n_sub, -1.0 + EPS, 1.0 - EPS)
        at = 0.5 * jnp.log((1.0 + cn) / (1.0 - cn))
        k = jnp.maximum(1.0 - P2, 1e-10) * at / n_sub
        kid2 = k * id2
        Cp = kid2 * (b2 * Yp - a2)
        Cx = kid2 * b2 * Yx
        Cr = kid2 * b2 * Yr

        o_ref[...] = Cp * p + Cx * xd + Cr * rl

    return pl.pallas_call(
        body,
        grid=(N_EBLK,),
        in_specs=[
            pl.BlockSpec((DIM, EBLK), lambda i: (0, i)),
            pl.BlockSpec((DIM, EBLK), lambda i: (0, i)),
            pl.BlockSpec((DIM, EBLK), lambda i: (0, i)),
            pl.BlockSpec((8, EBLK), lambda i: (8, i)),
        ],
        out_specs=pl.BlockSpec((DIM, EBLK), lambda i: (0, i)),
        out_shape=_sds((DIM, EKG_PAD)),
    )(pT, xdT, relx, relx)


def _tc_node(ego_items, item_cf, u_embed, g1, g2):
    nb_i = N_ITEMS // 1000
    nb = N_NODE // 1000

    def body(e_ref, cf_ref, u_ref, g1_ref, g2_ref, lo_ref, hi_ref):
        i = pl.program_id(0)

        @pl.when(i < nb_i)
        def _():
            e = e_ref[...]
            icf = cf_ref[...]
            dn = (((1,), (1,)), ((), ()))
            z = (lax.dot_general(e, g1_ref[...], dn,
                                 preferred_element_type=jnp.float32,
                                 precision=jax.lax.Precision.HIGHEST)
                 + lax.dot_general(icf, g2_ref[...], dn,
                                   preferred_element_type=jnp.float32,
                                   precision=jax.lax.Precision.HIGHEST))
            gi = jax.nn.sigmoid(z)
            fus = gi * e + (1.0 - gi) * icf
            lo_ref[...] = fus[:, :HALF]
            hi_ref[...] = fus[:, HALF:]

        @pl.when(i >= nb_i)
        def _():
            u = u_ref[...]
            lo_ref[...] = u[:, :HALF]
            hi_ref[...] = u[:, HALF:]

    return pl.pallas_call(
        body,
        grid=(nb,),
        in_specs=[
            pl.BlockSpec((1000, DIM), lambda i: (jnp.minimum(i, nb_i - 1), 0)),
            pl.BlockSpec((1000, DIM), lambda i: (jnp.minimum(i, nb_i - 1), 0)),
            pl.BlockSpec((1000, DIM), lambda i: (jnp.maximum(i - nb_i, 0), 0)),
            pl.BlockSpec((DIM, DIM), lambda i: (0, 0)),
            pl.BlockSpec((DIM, DIM), lambda i: (0, 0)),
        ],
        out_specs=[
            pl.BlockSpec((1000, HALF), lambda i: (i, 0)),
            pl.BlockSpec((1000, HALF), lambda i: (i, 0)),
        ],
        out_shape=(_sds((N_NODE, HALF)), _sds((N_NODE, HALF))),
    )(ego_items, item_cf, u_embed, g1, g2)


def _finalize_body(s2_ref, c2_ref, res_ref, new_ref, ro_ref):
    s = jnp.concatenate([s2_ref[0], s2_ref[1]], axis=1)
    cnt = c2_ref[0][:, :1] + c2_ref[1][:, :1]
    m = s / jnp.maximum(cnt, 1.0)
    e = _l2n(m)
    new_ref[...] = e
    ro_ref[...] = res_ref[...] + e


def _tc_fin(s2, c2, res, row_off, n_rows):
    nb = n_rows // 1000

    return pl.pallas_call(
        _finalize_body,
        grid=(nb,),
        in_specs=[
            pl.BlockSpec((NC, 1000, HALF), lambda i: (0, i + row_off, 0)),
            pl.BlockSpec((NC, 1000, CNT_W), lambda i: (0, i + row_off, 0)),
            pl.BlockSpec((1000, DIM), lambda i: (i, 0)),
        ],
        out_specs=[
            pl.BlockSpec((1000, DIM), lambda i: (i, 0)),
            pl.BlockSpec((1000, DIM), lambda i: (i, 0)),
        ],
        out_shape=(_sds((n_rows, DIM)), _sds((n_rows, DIM))),
    )(s2, c2, res)


def kernel(kg_edge_index, kg_edge_type, i2u_edge_index, entity_user_embed,
           relation_embed, items_embed_cf, gate1_w, gate2_w):
    src = kg_edge_index[0].astype(jnp.int32)
    dst = kg_edge_index[1].astype(jnp.int32)
    i2s = i2u_edge_index[0].astype(jnp.int32)
    i2d = i2u_edge_index[1].astype(jnp.int32)
    ktype = kg_edge_type.astype(jnp.int32)

    pad_kg = EKG_PAD - E_KG
    pad_2 = E2_PAD - E_I2U
    srcg4 = jnp.pad(src, (0, pad_kg)).reshape(NW, KG_CHG, CH)
    dstg4 = jnp.pad(dst, (0, pad_kg)).reshape(NW, KG_CHG, CH)
    src_s = jnp.pad(src, (0, pad_kg), constant_values=N_ENT)
    srcs4 = src_s.reshape(NS, KG_CHS // KI, KI, CH)
    srccnt4 = src_s.reshape(NW, KG_CHG // KI, KI, CH)
    s2g = jnp.pad(i2s, (0, pad_2))
    src5 = jnp.stack([s2g, s2g + N_NODE]).reshape(
        NC, NS, I2U_CHS // KI, KI, CH)
    d2_s = jnp.pad(i2d, (0, pad_2), constant_values=N_NODE)
    dst4 = d2_s.reshape(NS, I2U_CHS // KI, KI, CH)
    dstcnt4 = d2_s.reshape(NW, I2U_CHG // KI, KI, CH)
    relm = jnp.pad(ktype, (0, pad_kg)).reshape(1, EKG_PAD)
    rtx = jnp.zeros((72, 16), jnp.float32)
    rtx = rtx.at[:DIM, :N_REL].set(relation_embed.T)
    rtx = rtx.at[DIM, :N_REL].set(jnp.sum(relation_embed ** 2, axis=1))

    zeros_kg = jnp.zeros((ACC_KG, HALF), jnp.float32)
    zeros_2 = jnp.zeros((ACC_2, HALF), jnp.float32)
    zkg16 = jnp.zeros((ACC_KG, CNT_W), jnp.float32)
    z2_16 = jnp.zeros((ACC_2, CNT_W), jnp.float32)
    ones16 = jnp.ones((CH, CNT_W), jnp.float32)

    ckg, c2 = _sc_counts(srccnt4, dstcnt4, ones16, zkg16, z2_16)
    relx = _tc_relsel(relm, rtx)

    ego = entity_user_embed[:N_ENT]
    u_embed = entity_user_embed[N_ENT:]
    item_cf = items_embed_cf
    ent_res, user_res, item_res = ego, u_embed, item_cf

    for layer in range(2):
        P = _tc_expmap0(ego)
        n_lo, n_hi = _tc_node(ego[:N_ITEMS], item_cf, u_embed,
                              gate1_w[layer], gate2_w[layer])
        node2 = jnp.concatenate([n_lo, n_hi], axis=0)
        p_src, x_dst = _sc_gather_kg(P, ego, srcg4, dstg4)
        a2 = _sc_i2u(node2, src5, dst4, zeros_2)
        tsT = _tc_edge_math(p_src.T, x_dst.T, relx)
        ts2 = tsT.reshape(NC, HALF, EKG_PAD).transpose(0, 2, 1)
        kg2 = _sc_scatter_kg(ts2, srcs4, zeros_kg)
        ego, ent_res = _tc_fin(kg2, ckg, ent_res, 0, N_ENT)
        item_cf, item_res = _tc_fin(a2, c2, item_res, 0, N_ITEMS)
        u_embed, user_res = _tc_fin(a2, c2, user_res, N_ITEMS // 1000,
                                    N_USERS)

    return ent_res, user_res, item_res


# speedup vs baseline: 1.2525x; 1.0577x over previous
import jax
import jax.numpy as jnp
from jax import lax
from jax.experimental import pallas as pl
from jax.experimental.pallas import tpu as pltpu
from jax.experimental.pallas import tpu_sc as plsc

N_ENT = 50000
N_USERS = 10000
N_ITEMS = 20000
N_REL = 10
DIM = 64
HALF = 32
E_KG = 800000
E_I2U = 600000
EPS = 1e-7

NC = 2
NS = 16
NW = NC * NS
CH = 128

KG_CHG = 196
TILE_KG_G = KG_CHG * CH
EKG_PAD = NW * TILE_KG_G
KG_CHS = 2 * KG_CHG
TILE_KG_S = KG_CHS * CH

I2U_CHG = 147
TILE_I2U_G = I2U_CHG * CH
E2_PAD = NW * TILE_I2U_G
I2U_CHS = 2 * I2U_CHG
TILE_I2U_S = I2U_CHS * CH

ACC_KG = 50048
ZKG = ACC_KG // NS
N_NODE = N_ITEMS + N_USERS
ACC_2 = 30080
Z2 = ACC_2 // NS
CNT_W = 16

EBLK = 2048
N_EBLK = EKG_PAD // EBLK

KI = 7

_SC_MESH = dict(core_axis_name="c", subcore_axis_name="s")


def _sds(shape, dtype=jnp.float32):
    return jax.ShapeDtypeStruct(shape, dtype)


def _norm(x):
    return jnp.sqrt(jnp.sum(x * x, axis=-1, keepdims=True) + 1e-15)


def _proj(x):
    maxnorm = 1.0 - 1e-3
    n = _norm(x)
    return jnp.where(n > maxnorm, x / n * maxnorm, x)


def _l2n(x):
    return x / _norm(x)


def _sc_gather_kg(P, ego, srcg4, dstg4):
    def body(p_h, ego_h, src_h, dst_h, ps_out, xd_out,
             idx_s, idx_d, pa, ea, pb, eb,
             gsa, gsb, wsa, wsb):
        c = lax.axis_index("c")
        s = lax.axis_index("s")
        wid = s * NC + c
        base = wid * TILE_KG_G
        pltpu.sync_copy(src_h.at[wid], idx_s)
        pltpu.sync_copy(dst_h.at[wid], idx_d)

        def fire_g(j, rp, re, sem):
            pltpu.async_copy(p_h.at[idx_s.at[j]], rp, sem)
            pltpu.async_copy(ego_h.at[idx_d.at[j]], re, sem)

        def wait_g(rp, re, sem):
            pltpu.make_async_copy(p_h.at[pl.ds(0, CH)], rp, sem).wait()
            pltpu.make_async_copy(p_h.at[pl.ds(0, CH)], re, sem).wait()

        def fire_w(j, rp, re, sem):
            off = base + j * CH
            pltpu.async_copy(rp, ps_out.at[pl.ds(off, CH)], sem)
            pltpu.async_copy(re, xd_out.at[pl.ds(off, CH)], sem)

        def wait_w(rp, re, sem):
            pltpu.make_async_copy(rp, ps_out.at[pl.ds(0, CH)], sem).wait()
            pltpu.make_async_copy(re, xd_out.at[pl.ds(0, CH)], sem).wait()

        fire_g(0, pa, ea, gsa)
        fire_g(1, pb, eb, gsb)
        n_it = KG_CHG // 2

        def step(o, carry):
            a = 2 * o
            wait_g(pa, ea, gsa)
            fire_w(a, pa, ea, wsa)
            wait_g(pb, eb, gsb)
            fire_w(a + 1, pb, eb, wsb)
            wait_w(pa, ea, wsa)

            @pl.when(o < n_it - 1)
            def _():
                fire_g(a + 2, pa, ea, gsa)

            wait_w(pb, eb, wsb)

            @pl.when(o < n_it - 1)
            def _():
                fire_g(a + 3, pb, eb, gsb)

            return carry

        lax.fori_loop(0, n_it, step, 0)

    return pl.kernel(
        body,
        out_type=(_sds((EKG_PAD, DIM)), _sds((EKG_PAD, DIM))),
        mesh=plsc.VectorSubcoreMesh(**_SC_MESH),
        compiler_params=pltpu.CompilerParams(use_tc_tiling_on_sc=False),
        scratch_types=[
            pltpu.VMEM((KG_CHG, CH), jnp.int32),
            pltpu.VMEM((KG_CHG, CH), jnp.int32),
            pltpu.VMEM((CH, DIM), jnp.float32),
            pltpu.VMEM((CH, DIM), jnp.float32),
            pltpu.VMEM((CH, DIM), jnp.float32),
            pltpu.VMEM((CH, DIM), jnp.float32),
            pltpu.SemaphoreType.DMA,
            pltpu.SemaphoreType.DMA,
            pltpu.SemaphoreType.DMA,
            pltpu.SemaphoreType.DMA,
        ],
    )(P, ego, srcg4, dstg4)


def _sc_scatter_kg(ts2, srcs4, zeros_kg):
    def body(ts_h, src_h, z_h, out_h, idx0, idx1, r0, r1, acc,
             is0, is1, rs0, rs1):
        c = lax.axis_index("c")
        s = lax.axis_index("s")
        pltpu.sync_copy(z_h.at[pl.ds(s * ZKG, ZKG)],
                        acc.at[pl.ds(s * ZKG, ZKG)])
        plsc.subcore_barrier()
        base = s * TILE_KG_S
        rbuf = (r0, r1)
        rsem = (rs0, rs1)

        def proc_group(g, idx_v):
            off0 = base + g * KI * CH
            pltpu.async_copy(ts_h.at[c, pl.ds(off0, CH)], r0, rs0)
            for kk in range(KI):
                cur, csem = rbuf[kk % 2], rsem[kk % 2]
                if kk < KI - 1:
                    off = base + (g * KI + kk + 1) * CH
                    pltpu.async_copy(ts_h.at[c, pl.ds(off, CH)],
                                     rbuf[(kk + 1) % 2], rsem[(kk + 1) % 2])
                pltpu.make_async_copy(ts_h.at[c, pl.ds(0, CH)],
                                      cur, csem).wait()
                pltpu.sync_copy(cur, acc.at[idx_v.at[kk]], add=True)

        n_pair = KG_CHS // KI // 2
        pltpu.async_copy(src_h.at[s, 0], idx0, is0)

        def step(m, carry):
            pltpu.make_async_copy(src_h.at[s, 0], idx0, is0).wait()
            pltpu.async_copy(src_h.at[s, 2 * m + 1], idx1, is1)
            proc_group(2 * m, idx0)
            pltpu.make_async_copy(src_h.at[s, 0], idx1, is1).wait()

            @pl.when(m < n_pair - 1)
            def _():
                pltpu.async_copy(src_h.at[s, 2 * m + 2], idx0, is0)

            proc_group(2 * m + 1, idx1)
            return carry

        lax.fori_loop(0, n_pair, step, 0)
        plsc.subcore_barrier()
        pltpu.sync_copy(acc.at[pl.ds(s * ZKG, ZKG)],
                        out_h.at[c, pl.ds(s * ZKG, ZKG)])

    return pl.kernel(
        body,
        out_type=_sds((NC, ACC_KG, HALF)),
        mesh=plsc.VectorSubcoreMesh(**_SC_MESH),
        compiler_params=pltpu.CompilerParams(use_tc_tiling_on_sc=False),
        scratch_types=[
            pltpu.VMEM((KI, CH), jnp.int32),
            pltpu.VMEM((KI, CH), jnp.int32),
            pltpu.VMEM((CH, HALF), jnp.float32),
            pltpu.VMEM((CH, HALF), jnp.float32),
            pltpu.VMEM_SHARED((ACC_KG, HALF), jnp.float32),
            pltpu.SemaphoreType.DMA,
            pltpu.SemaphoreType.DMA,
            pltpu.SemaphoreType.DMA,
            pltpu.SemaphoreType.DMA,
        ],
    )(ts2, srcs4, zeros_kg)


def _sc_i2u(node2, src5, dst4, zeros_2):
    def body(n_h, src_h, dst_h, z_h, out_h, is0v, is1v, id0v, id1v,
             r0, r1, acc, is0, is1, rs0, rs1):
        c = lax.axis_index("c")
        s = lax.axis_index("s")
        pltpu.sync_copy(z_h.at[pl.ds(s * Z2, Z2)], acc.at[pl.ds(s * Z2, Z2)])
        plsc.subcore_barrier()
        rbuf = (r0, r1)
        rsem = (rs0, rs1)

        def proc_group(sv, dv):
            pltpu.async_copy(n_h.at[sv.at[0]], r0, rs0)
            for kk in range(KI):
                cur, csem = rbuf[kk % 2], rsem[kk % 2]
                if kk < KI - 1:
                    pltpu.async_copy(n_h.at[sv.at[kk + 1]],
                                     rbuf[(kk + 1) % 2], rsem[(kk + 1) % 2])
                pltpu.make_async_copy(n_h.at[pl.ds(0, CH)], cur, csem).wait()
                pltpu.sync_copy(cur, acc.at[dv.at[kk]], add=True)

        n_pair = I2U_CHS // KI // 2
        pltpu.async_copy(src_h.at[c, s, 0], is0v, is0)
        pltpu.async_copy(dst_h.at[s, 0], id0v, is0)

        def step(m, carry):
            pltpu.make_async_copy(src_h.at[c, s, 0], is0v, is0).wait()
            pltpu.make_async_copy(dst_h.at[s, 0], id0v, is0).wait()
            pltpu.async_copy(src_h.at[c, s, 2 * m + 1], is1v, is1)
            pltpu.async_copy(dst_h.at[s, 2 * m + 1], id1v, is1)
            proc_group(is0v, id0v)
            pltpu.make_async_copy(src_h.at[c, s, 0], is1v, is1).wait()
            pltpu.make_async_copy(dst_h.at[s, 0], id1v, is1).wait()

            @pl.when(m < n_pair - 1)
            def _():
                pltpu.async_copy(src_h.at[c, s, 2 * m + 2], is0v, is0)
                pltpu.async_copy(dst_h.at[s, 2 * m + 2], id0v, is0)

            proc_group(is1v, id1v)
            return carry

        lax.fori_loop(0, n_pair, step, 0)
        plsc.subcore_barrier()
        pltpu.sync_copy(acc.at[pl.ds(s * Z2, Z2)],
                        out_h.at[c, pl.ds(s * Z2, Z2)])

    return pl.kernel(
        body,
        out_type=_sds((NC, ACC_2, HALF)),
        mesh=plsc.VectorSubcoreMesh(**_SC_MESH),
        compiler_params=pltpu.CompilerParams(use_tc_tiling_on_sc=False),
        scratch_types=[
            pltpu.VMEM((KI, CH), jnp.int32),
            pltpu.VMEM((KI, CH), jnp.int32),
            pltpu.VMEM((KI, CH), jnp.int32),
            pltpu.VMEM((KI, CH), jnp.int32),
            pltpu.VMEM((CH, HALF), jnp.float32),
            pltpu.VMEM((CH, HALF), jnp.float32),
            pltpu.VMEM_SHARED((ACC_2, HALF), jnp.float32),
            pltpu.SemaphoreType.DMA,
            pltpu.SemaphoreType.DMA,
            pltpu.SemaphoreType.DMA,
            pltpu.SemaphoreType.DMA,
        ],
    )(node2, src5, dst4, zeros_2)


def _sc_counts(srccnt4, dstcnt4, ones_h, zkg16, z2_16):
    def body(src_h, dst_h, on_h, zk_h, z2_h, okg, o2,
             idx_kg, idx_2, ones_v, acck, acc2):
        c = lax.axis_index("c")
        s = lax.axis_index("s")
        wid = s * NC + c
        pltpu.sync_copy(zk_h.at[pl.ds(s * ZKG, ZKG)],
                        acck.at[pl.ds(s * ZKG, ZKG)])
        pltpu.sync_copy(z2_h.at[pl.ds(s * Z2, Z2)],
                        acc2.at[pl.ds(s * Z2, Z2)])
        pltpu.sync_copy(on_h, ones_v)
        plsc.subcore_barrier()

        def chunk_kg(o, carry):
            pltpu.sync_copy(src_h.at[wid, o], idx_kg)
            for kk in range(KI):
                pltpu.sync_copy(ones_v, acck.at[idx_kg.at[kk]], add=True)
            return carry

        lax.fori_loop(0, KG_CHG // KI, chunk_kg, 0)

        def chunk_2(o, carry):
            pltpu.sync_copy(dst_h.at[wid, o], idx_2)
            for kk in range(KI):
                pltpu.sync_copy(ones_v, acc2.at[idx_2.at[kk]], add=True)
            return carry

        lax.fori_loop(0, I2U_CHG // KI, chunk_2, 0)
        plsc.subcore_barrier()
        pltpu.sync_copy(acck.at[pl.ds(s * ZKG, ZKG)],
                        okg.at[c, pl.ds(s * ZKG, ZKG)])
        pltpu.sync_copy(acc2.at[pl.ds(s * Z2, Z2)],
                        o2.at[c, pl.ds(s * Z2, Z2)])

    return pl.kernel(
        body,
        out_type=(_sds((NC, ACC_KG, CNT_W)), _sds((NC, ACC_2, CNT_W))),
        mesh=plsc.VectorSubcoreMesh(**_SC_MESH),
        compiler_params=pltpu.CompilerParams(use_tc_tiling_on_sc=False),
        scratch_types=[
            pltpu.VMEM((KI, CH), jnp.int32),
            pltpu.VMEM((KI, CH), jnp.int32),
            pltpu.VMEM((CH, CNT_W), jnp.float32),
            pltpu.VMEM_SHARED((ACC_KG, CNT_W), jnp.float32),
            pltpu.VMEM_SHARED((ACC_2, CNT_W), jnp.float32),
        ],
    )(srccnt4, dstcnt4, ones_h, zkg16, z2_16)


def _tc_expmap0(ego):
    def body(x_ref, o_ref):
        x = x_ref[...]
        n = _norm(x)
        o_ref[...] = _proj(jnp.tanh(n) * x / n)

    return pl.pallas_call(
        body,
        grid=(25,),
        in_specs=[pl.BlockSpec((2000, DIM), lambda i: (i, 0))],
        out_specs=pl.BlockSpec((2000, DIM), lambda i: (i, 0)),
        out_shape=_sds((N_ENT, DIM)),
    )(ego)


def _tc_relsel(relm, rtx):
    def body(rm_ref, rt_ref, o_ref):
        rm = rm_ref[...]
        oh = (lax.broadcasted_iota(jnp.int32, (16, EBLK), 0)
              == (rm + 2)).astype(jnp.float32)
        o_ref[...] = jax.lax.dot_general(
            rt_ref[...], oh, (((1,), (0,)), ((), ())),
            preferred_element_type=jnp.float32,
            precision=jax.lax.Precision.HIGHEST)

    return pl.pallas_call(
        body,
        grid=(N_EBLK,),
        in_specs=[
            pl.BlockSpec((1, EBLK), lambda i: (0, i)),
            pl.BlockSpec((72, 16), lambda i: (0, 0)),
        ],
        out_specs=pl.BlockSpec((72, EBLK), lambda i: (0, i)),
        out_shape=_sds((72, EKG_PAD)),
    )(relm, rtx)


def _tc_edge_math(pT, xdT, relx):
    def body(p_ref, x_ref, r_ref, rn_ref, o_ref):
        p = p_ref[...]
        xd = x_ref[...]
        rl = r_ref[...]
        mx = 1.0 - 1e-3

        def rsum(v):
            return jnp.sum(v, axis=0, keepdims=True)

        P2 = rsum(p * p)
        X2 = rsum(xd * xd)
        R2 = rn_ref[0:1, :]
        PX = rsum(p * xd)
        PR = rsum(p * rl)
        XR = rsum(xd * rl)

        lam = 2.0 / jnp.maximum(1.0 - P2, 1e-10)

        def expm_coeffs(U2, PU):
            n_u = jnp.sqrt(U2 + 1e-15)
            t = jnp.tanh(lam * n_u / 2.0)
            sc = t / n_u
            s2 = sc * sc * U2
            dps = sc * PU
            a = 1.0 + 2.0 * dps + s2
            b = 1.0 - P2
            iden = 1.0 / jnp.maximum(1.0 + 2.0 * dps + P2 * s2, 1e-10)
            s_m = jnp.maximum(
                iden * iden * (a * a * P2 + 2.0 * a * b * dps + b * b * s2),
                0.0)
            n_m = jnp.sqrt(s_m + 1e-15)
            f = jnp.where(n_m > mx, mx / n_m, 1.0)
            return f * iden * a, f * iden * b * sc, f * f * s_m

        Ap, Ax, SA = expm_coeffs(X2, PX)
        Bp, Br, SB = expm_coeffs(R2, PR)

        dab = (Ap * Bp * P2 + Ap * Br * PR + Ax * Bp * PX + Ax * Br * XR)
        a1 = 1.0 + 2.0 * dab + SB
        b1 = 1.0 - SA
        id1 = 1.0 / jnp.maximum(1.0 + 2.0 * dab + SA * SB, 1e-10)
        s_m1 = jnp.maximum(
            id1 * id1 * (a1 * a1 * SA + 2.0 * a1 * b1 * dab + b1 * b1 * SB),
            0.0)
        n_m1 = jnp.sqrt(s_m1 + 1e-15)
        f1 = jnp.where(n_m1 > mx, mx / n_m1, 1.0)
        g1 = f1 * id1
        Yp = g1 * (a1 * Ap + b1 * Bp)
        Yx = g1 * a1 * Ax
        Yr = g1 * b1 * Br
        SY = f1 * f1 * s_m1

        dpy = Yp * P2 + Yx * PX + Yr * PR
        a2 = 1.0 - 2.0 * dpy + SY
        b2 = 1.0 - P2
        id2 = 1.0 / jnp.maximum(1.0 - 2.0 * dpy + P2 * SY, 1e-10)
        s_sub = jnp.maximum(
            id2 * id2 * (a2 * a2 * P2 - 2.0 * a2 * b2 * dpy + b2 * b2 * SY),
            0.0)
        n_sub = jnp.sqrt(s_sub + 1e-15)

        cn = jnp.clip(n_sub, -1.0 + EPS, 1.0 - EPS)
        at = 0.5 * jnp.log((1.0 + cn) / (1.0 - cn))
        k = jnp.maximum(1.0 - P2, 1e-10) * at / n_sub
        kid2 = k * id2
        Cp = kid2 * (b2 * Yp - a2)
        Cx = kid2 * b2 * Yx
        Cr = kid2 * b2 * Yr

        o_ref[...] = Cp * p + Cx * xd + Cr * rl

    return pl.pallas_call(
        body,
        grid=(N_EBLK,),
        in_specs=[
            pl.BlockSpec((DIM, EBLK), lambda i: (0, i)),
            pl.BlockSpec((DIM, EBLK), lambda i: (0, i)),
            pl.BlockSpec((DIM, EBLK), lambda i: (0, i)),
            pl.BlockSpec((8, EBLK), lambda i: (8, i)),
        ],
        out_specs=pl.BlockSpec((DIM, EBLK), lambda i: (0, i)),
        out_shape=_sds((DIM, EKG_PAD)),
    )(pT, xdT, relx, relx)


def _tc_node(ego_items, item_cf, u_embed, g1, g2):
    nb_i = N_ITEMS // 1000
    nb = N_NODE // 1000

    def body(e_ref, cf_ref, u_ref, g1_ref, g2_ref, lo_ref, hi_ref):
        i = pl.program_id(0)

        @pl.when(i < nb_i)
        def _():
            e = e_ref[...]
            icf = cf_ref[...]
            dn = (((1,), (1,)), ((), ()))
            z = (lax.dot_general(e, g1_ref[...], dn,
                                 preferred_element_type=jnp.float32,
                                 precision=jax.lax.Precision.HIGHEST)
                 + lax.dot_general(icf, g2_ref[...], dn,
                                   preferred_element_type=jnp.float32,
                                   precision=jax.lax.Precision.HIGHEST))
            gi = jax.nn.sigmoid(z)
            fus = gi * e + (1.0 - gi) * icf
            lo_ref[...] = fus[:, :HALF]
            hi_ref[...] = fus[:, HALF:]

        @pl.when(i >= nb_i)
        def _():
            u = u_ref[...]
            lo_ref[...] = u[:, :HALF]
            hi_ref[...] = u[:, HALF:]

    return pl.pallas_call(
        body,
        grid=(nb,),
        in_specs=[
            pl.BlockSpec((1000, DIM), lambda i: (jnp.minimum(i, nb_i - 1), 0)),
            pl.BlockSpec((1000, DIM), lambda i: (jnp.minimum(i, nb_i - 1), 0)),
            pl.BlockSpec((1000, DIM), lambda i: (jnp.maximum(i - nb_i, 0), 0)),
            pl.BlockSpec((DIM, DIM), lambda i: (0, 0)),
            pl.BlockSpec((DIM, DIM), lambda i: (0, 0)),
        ],
        out_specs=[
            pl.BlockSpec((1000, HALF), lambda i: (i, 0)),
            pl.BlockSpec((1000, HALF), lambda i: (i, 0)),
        ],
        out_shape=(_sds((N_NODE, HALF)), _sds((N_NODE, HALF))),
    )(ego_items, item_cf, u_embed, g1, g2)


def _finalize_body(s2_ref, c2_ref, res_ref, new_ref, ro_ref):
    s = jnp.concatenate([s2_ref[0], s2_ref[1]], axis=1)
    cnt = c2_ref[0][:, :1] + c2_ref[1][:, :1]
    m = s / jnp.maximum(cnt, 1.0)
    e = _l2n(m)
    new_ref[...] = e
    ro_ref[...] = res_ref[...] + e


def _tc_fin(s2, c2, res, row_off, n_rows):
    nb = n_rows // 1000

    return pl.pallas_call(
        _finalize_body,
        grid=(nb,),
        in_specs=[
            pl.BlockSpec((NC, 1000, HALF), lambda i: (0, i + row_off, 0)),
            pl.BlockSpec((NC, 1000, CNT_W), lambda i: (0, i + row_off, 0)),
            pl.BlockSpec((1000, DIM), lambda i: (i, 0)),
        ],
        out_specs=[
            pl.BlockSpec((1000, DIM), lambda i: (i, 0)),
            pl.BlockSpec((1000, DIM), lambda i: (i, 0)),
        ],
        out_shape=(_sds((n_rows, DIM)), _sds((n_rows, DIM))),
    )(s2, c2, res)


def kernel(kg_edge_index, kg_edge_type, i2u_edge_index, entity_user_embed,
           relation_embed, items_embed_cf, gate1_w, gate2_w):
    src = kg_edge_index[0].astype(jnp.int32)
    dst = kg_edge_index[1].astype(jnp.int32)
    i2s = i2u_edge_index[0].astype(jnp.int32)
    i2d = i2u_edge_index[1].astype(jnp.int32)
    ktype = kg_edge_type.astype(jnp.int32)

    pad_kg = EKG_PAD - E_KG
    pad_2 = E2_PAD - E_I2U
    srcg4 = jnp.pad(src, (0, pad_kg)).reshape(NW, KG_CHG, CH)
    dstg4 = jnp.pad(dst, (0, pad_kg)).reshape(NW, KG_CHG, CH)
    src_s = jnp.pad(src, (0, pad_kg), constant_values=N_ENT)
    srcs4 = src_s.reshape(NS, KG_CHS // KI, KI, CH)
    srccnt4 = src_s.reshape(NW, KG_CHG // KI, KI, CH)
    s2g = jnp.pad(i2s, (0, pad_2))
    src5 = jnp.stack([s2g, s2g + N_NODE]).reshape(
        NC, NS, I2U_CHS // KI, KI, CH)
    d2_s = jnp.pad(i2d, (0, pad_2), constant_values=N_NODE)
    dst4 = d2_s.reshape(NS, I2U_CHS // KI, KI, CH)
    dstcnt4 = d2_s.reshape(NW, I2U_CHG // KI, KI, CH)
    relm = jnp.pad(ktype, (0, pad_kg)).reshape(1, EKG_PAD)
    rtx = jnp.zeros((72, 16), jnp.float32)
    rtx = rtx.at[:DIM, :N_REL].set(relation_embed.T)
    rtx = rtx.at[DIM, :N_REL].set(jnp.sum(relation_embed ** 2, axis=1))

    zeros_kg = jnp.zeros((ACC_KG, HALF), jnp.float32)
    zeros_2 = jnp.zeros((ACC_2, HALF), jnp.float32)
    zkg16 = jnp.zeros((ACC_KG, CNT_W), jnp.float32)
    z2_16 = jnp.zeros((ACC_2, CNT_W), jnp.float32)
    ones16 = jnp.ones((CH, CNT_W), jnp.float32)

    ckg, c2 = _sc_counts(srccnt4, dstcnt4, ones16, zkg16, z2_16)
    relx = _tc_relsel(relm, rtx)

    ego = entity_user_embed[:N_ENT]
    u_embed = entity_user_embed[N_ENT:]
    item_cf = items_embed_cf
    ent_res, user_res, item_res = ego, u_embed, item_cf

    for layer in range(2):
        P = _tc_expmap0(ego)
        n_lo, n_hi = _tc_node(ego[:N_ITEMS], item_cf, u_embed,
                              gate1_w[layer], gate2_w[layer])
        node2 = jnp.concatenate([n_lo, n_hi], axis=0)
        p_src, x_dst = _sc_gather_kg(P, ego, srcg4, dstg4)
        a2 = _sc_i2u(node2, src5, dst4, zeros_2)
        tsT = _tc_edge_math(p_src.T, x_dst.T, relx)
        ts2 = tsT.reshape(NC, HALF, EKG_PAD).transpose(0, 2, 1)
        kg2 = _sc_scatter_kg(ts2, srcs4, zeros_kg)
        ego, ent_res = _tc_fin(kg2, ckg, ent_res, 0, N_ENT)
        item_cf, item_res = _tc_fin(a2, c2, item_res, 0, N_ITEMS)
        u_embed, user_res = _tc_fin(a2, c2, user_res, N_ITEMS // 1000,
                                    N_USERS)

    return ent_res, user_res, item_res


# speedup vs baseline: 1.2983x; 1.0366x over previous
import jax
import jax.numpy as jnp
from jax import lax
from jax.experimental import pallas as pl
from jax.experimental.pallas import tpu as pltpu
from jax.experimental.pallas import tpu_sc as plsc

N_ENT = 50000
N_USERS = 10000
N_ITEMS = 20000
N_REL = 10
DIM = 64
HALF = 32
E_KG = 800000
E_I2U = 600000
EPS = 1e-7

NC = 2
NS = 16
NW = NC * NS
CH = 128

KG_CHG = 196
TILE_KG_G = KG_CHG * CH
EKG_PAD = NW * TILE_KG_G
KG_CHS = 2 * KG_CHG
TILE_KG_S = KG_CHS * CH

I2U_CHG = 147
TILE_I2U_G = I2U_CHG * CH
E2_PAD = NW * TILE_I2U_G
I2U_CHS = 2 * I2U_CHG
TILE_I2U_S = I2U_CHS * CH

ACC_KG = 50048
ZKG = ACC_KG // NS
N_NODE = N_ITEMS + N_USERS
ACC_2 = 30080
Z2 = ACC_2 // NS
CNT_W = 16

EBLK = 2048
N_EBLK = EKG_PAD // EBLK

KI = 7

_SC_MESH = dict(core_axis_name="c", subcore_axis_name="s")


def _sds(shape, dtype=jnp.float32):
    return jax.ShapeDtypeStruct(shape, dtype)


def _norm(x):
    return jnp.sqrt(jnp.sum(x * x, axis=-1, keepdims=True) + 1e-15)


def _proj(x):
    maxnorm = 1.0 - 1e-3
    n = _norm(x)
    return jnp.where(n > maxnorm, x / n * maxnorm, x)


def _l2n(x):
    return x / _norm(x)


def _sc_gather_kg(P, ego, srcg4, dstg4):
    def body(p_h, ego_h, src_h, dst_h, ps_out, xd_out,
             idx_s, idx_d, pa, ea, pb, eb,
             gsa, gsb, wsa, wsb):
        c = lax.axis_index("c")
        s = lax.axis_index("s")
        wid = s * NC + c
        base = wid * TILE_KG_G
        pltpu.sync_copy(src_h.at[wid], idx_s)
        pltpu.sync_copy(dst_h.at[wid], idx_d)

        def fire_g(j, rp, re, sem):
            pltpu.async_copy(p_h.at[idx_s.at[j]], rp, sem)
            pltpu.async_copy(ego_h.at[idx_d.at[j]], re, sem)

        def wait_g(rp, re, sem):
            pltpu.make_async_copy(p_h.at[pl.ds(0, CH)], rp, sem).wait()
            pltpu.make_async_copy(p_h.at[pl.ds(0, CH)], re, sem).wait()

        def fire_w(j, rp, re, sem):
            off = base + j * CH
            pltpu.async_copy(rp, ps_out.at[pl.ds(off, CH)], sem)
            pltpu.async_copy(re, xd_out.at[pl.ds(off, CH)], sem)

        def wait_w(rp, re, sem):
            pltpu.make_async_copy(rp, ps_out.at[pl.ds(0, CH)], sem).wait()
            pltpu.make_async_copy(re, xd_out.at[pl.ds(0, CH)], sem).wait()

        fire_g(0, pa, ea, gsa)
        fire_g(1, pb, eb, gsb)
        n_it = KG_CHG // 2

        def step(o, carry):
            a = 2 * o
            wait_g(pa, ea, gsa)
            fire_w(a, pa, ea, wsa)
            wait_g(pb, eb, gsb)
            fire_w(a + 1, pb, eb, wsb)
            wait_w(pa, ea, wsa)

            @pl.when(o < n_it - 1)
            def _():
                fire_g(a + 2, pa, ea, gsa)

            wait_w(pb, eb, wsb)

            @pl.when(o < n_it - 1)
            def _():
                fire_g(a + 3, pb, eb, gsb)

            return carry

        lax.fori_loop(0, n_it, step, 0)

    return pl.kernel(
        body,
        out_type=(_sds((EKG_PAD, DIM)), _sds((EKG_PAD, DIM))),
        mesh=plsc.VectorSubcoreMesh(**_SC_MESH),
        compiler_params=pltpu.CompilerParams(use_tc_tiling_on_sc=False),
        scratch_types=[
            pltpu.VMEM((KG_CHG, CH), jnp.int32),
            pltpu.VMEM((KG_CHG, CH), jnp.int32),
            pltpu.VMEM((CH, DIM), jnp.float32),
            pltpu.VMEM((CH, DIM), jnp.float32),
            pltpu.VMEM((CH, DIM), jnp.float32),
            pltpu.VMEM((CH, DIM), jnp.float32),
            pltpu.SemaphoreType.DMA,
            pltpu.SemaphoreType.DMA,
            pltpu.SemaphoreType.DMA,
            pltpu.SemaphoreType.DMA,
        ],
    )(P, ego, srcg4, dstg4)


def _sc_scatter_kg(ts2, srcs4, zeros_kg):
    def body(ts_h, src_h, z_h, out_h, idx0, idx1, r0, r1, acc,
             is0, is1, rs0, rs1):
        c = lax.axis_index("c")
        s = lax.axis_index("s")
        pltpu.sync_copy(z_h.at[pl.ds(s * ZKG, ZKG)],
                        acc.at[pl.ds(s * ZKG, ZKG)])
        plsc.subcore_barrier()
        base = s * TILE_KG_S
        rbuf = (r0, r1)
        rsem = (rs0, rs1)

        def proc_group(g, idx_v):
            off0 = base + g * KI * CH
            pltpu.async_copy(ts_h.at[c, pl.ds(off0, CH)], r0, rs0)
            for kk in range(KI):
                cur, csem = rbuf[kk % 2], rsem[kk % 2]
                if kk < KI - 1:
                    off = base + (g * KI + kk + 1) * CH
                    pltpu.async_copy(ts_h.at[c, pl.ds(off, CH)],
                                     rbuf[(kk + 1) % 2], rsem[(kk + 1) % 2])
                pltpu.make_async_copy(ts_h.at[c, pl.ds(0, CH)],
                                      cur, csem).wait()
                pltpu.sync_copy(cur, acc.at[idx_v.at[kk]], add=True)

        n_pair = KG_CHS // KI // 2
        pltpu.async_copy(src_h.at[s, 0], idx0, is0)

        def step(m, carry):
            pltpu.make_async_copy(src_h.at[s, 0], idx0, is0).wait()
            pltpu.async_copy(src_h.at[s, 2 * m + 1], idx1, is1)
            proc_group(2 * m, idx0)
            pltpu.make_async_copy(src_h.at[s, 0], idx1, is1).wait()

            @pl.when(m < n_pair - 1)
            def _():
                pltpu.async_copy(src_h.at[s, 2 * m + 2], idx0, is0)

            proc_group(2 * m + 1, idx1)
            return carry

        lax.fori_loop(0, n_pair, step, 0)
        plsc.subcore_barrier()
        pltpu.sync_copy(acc.at[pl.ds(s * ZKG, ZKG)],
                        out_h.at[c, pl.ds(s * ZKG, ZKG)])

    return pl.kernel(
        body,
        out_type=_sds((NC, ACC_KG, HALF)),
        mesh=plsc.VectorSubcoreMesh(**_SC_MESH),
        compiler_params=pltpu.CompilerParams(use_tc_tiling_on_sc=False),
        scratch_types=[
            pltpu.VMEM((KI, CH), jnp.int32),
            pltpu.VMEM((KI, CH), jnp.int32),
            pltpu.VMEM((CH, HALF), jnp.float32),
            pltpu.VMEM((CH, HALF), jnp.float32),
            pltpu.VMEM_SHARED((ACC_KG, HALF), jnp.float32),
            pltpu.SemaphoreType.DMA,
            pltpu.SemaphoreType.DMA,
            pltpu.SemaphoreType.DMA,
            pltpu.SemaphoreType.DMA,
        ],
    )(ts2, srcs4, zeros_kg)


def _sc_i2u(node2, src5, dst4, zeros_2):
    def body(n_h, src_h, dst_h, z_h, out_h, is0v, is1v, id0v, id1v,
             r0, r1, acc, is0, is1, rs0, rs1):
        c = lax.axis_index("c")
        s = lax.axis_index("s")
        pltpu.sync_copy(z_h.at[pl.ds(s * Z2, Z2)], acc.at[pl.ds(s * Z2, Z2)])
        plsc.subcore_barrier()
        rbuf = (r0, r1)
        rsem = (rs0, rs1)

        def proc_group(sv, dv):
            pltpu.async_copy(n_h.at[sv.at[0]], r0, rs0)
            for kk in range(KI):
                cur, csem = rbuf[kk % 2], rsem[kk % 2]
                if kk < KI - 1:
                    pltpu.async_copy(n_h.at[sv.at[kk + 1]],
                                     rbuf[(kk + 1) % 2], rsem[(kk + 1) % 2])
                pltpu.make_async_copy(n_h.at[pl.ds(0, CH)], cur, csem).wait()
                pltpu.sync_copy(cur, acc.at[dv.at[kk]], add=True)

        n_pair = I2U_CHS // KI // 2
        pltpu.async_copy(src_h.at[c, s, 0], is0v, is0)
        pltpu.async_copy(dst_h.at[s, 0], id0v, is0)

        def step(m, carry):
            pltpu.make_async_copy(src_h.at[c, s, 0], is0v, is0).wait()
            pltpu.make_async_copy(dst_h.at[s, 0], id0v, is0).wait()
            pltpu.async_copy(src_h.at[c, s, 2 * m + 1], is1v, is1)
            pltpu.async_copy(dst_h.at[s, 2 * m + 1], id1v, is1)
            proc_group(is0v, id0v)
            pltpu.make_async_copy(src_h.at[c, s, 0], is1v, is1).wait()
            pltpu.make_async_copy(dst_h.at[s, 0], id1v, is1).wait()

            @pl.when(m < n_pair - 1)
            def _():
                pltpu.async_copy(src_h.at[c, s, 2 * m + 2], is0v, is0)
                pltpu.async_copy(dst_h.at[s, 2 * m + 2], id0v, is0)

            proc_group(is1v, id1v)
            return carry

        lax.fori_loop(0, n_pair, step, 0)
        plsc.subcore_barrier()
        pltpu.sync_copy(acc.at[pl.ds(s * Z2, Z2)],
                        out_h.at[c, pl.ds(s * Z2, Z2)])

    return pl.kernel(
        body,
        out_type=_sds((NC, ACC_2, HALF)),
        mesh=plsc.VectorSubcoreMesh(**_SC_MESH),
        compiler_params=pltpu.CompilerParams(use_tc_tiling_on_sc=False),
        scratch_types=[
            pltpu.VMEM((KI, CH), jnp.int32),
            pltpu.VMEM((KI, CH), jnp.int32),
            pltpu.VMEM((KI, CH), jnp.int32),
            pltpu.VMEM((KI, CH), jnp.int32),
            pltpu.VMEM((CH, HALF), jnp.float32),
            pltpu.VMEM((CH, HALF), jnp.float32),
            pltpu.VMEM_SHARED((ACC_2, HALF), jnp.float32),
            pltpu.SemaphoreType.DMA,
            pltpu.SemaphoreType.DMA,
            pltpu.SemaphoreType.DMA,
            pltpu.SemaphoreType.DMA,
        ],
    )(node2, src5, dst4, zeros_2)


def _sc_counts(srccnt4, dstcnt4, ones_h, zkg16, z2_16):
    def body(src_h, dst_h, on_h, zk_h, z2_h, okg, o2,
             idx_kg, idx_2, ones_v, acck, acc2):
        c = lax.axis_index("c")
        s = lax.axis_index("s")
        wid = s * NC + c
        pltpu.sync_copy(zk_h.at[pl.ds(s * ZKG, ZKG)],
                        acck.at[pl.ds(s * ZKG, ZKG)])
        pltpu.sync_copy(z2_h.at[pl.ds(s * Z2, Z2)],
                        acc2.at[pl.ds(s * Z2, Z2)])
        pltpu.sync_copy(on_h, ones_v)
        plsc.subcore_barrier()

        def chunk_kg(o, carry):
            pltpu.sync_copy(src_h.at[wid, o], idx_kg)
            for kk in range(KI):
                pltpu.sync_copy(ones_v, acck.at[idx_kg.at[kk]], add=True)
            return carry

        lax.fori_loop(0, KG_CHG // KI, chunk_kg, 0)

        def chunk_2(o, carry):
            pltpu.sync_copy(dst_h.at[wid, o], idx_2)
            for kk in range(KI):
                pltpu.sync_copy(ones_v, acc2.at[idx_2.at[kk]], add=True)
            return carry

        lax.fori_loop(0, I2U_CHG // KI, chunk_2, 0)
        plsc.subcore_barrier()
        pltpu.sync_copy(acck.at[pl.ds(s * ZKG, ZKG)],
                        okg.at[c, pl.ds(s * ZKG, ZKG)])
        pltpu.sync_copy(acc2.at[pl.ds(s * Z2, Z2)],
                        o2.at[c, pl.ds(s * Z2, Z2)])

    return pl.kernel(
        body,
        out_type=(_sds((NC, ACC_KG, CNT_W)), _sds((NC, ACC_2, CNT_W))),
        mesh=plsc.VectorSubcoreMesh(**_SC_MESH),
        compiler_params=pltpu.CompilerParams(use_tc_tiling_on_sc=False),
        scratch_types=[
            pltpu.VMEM((KI, CH), jnp.int32),
            pltpu.VMEM((KI, CH), jnp.int32),
            pltpu.VMEM((CH, CNT_W), jnp.float32),
            pltpu.VMEM_SHARED((ACC_KG, CNT_W), jnp.float32),
            pltpu.VMEM_SHARED((ACC_2, CNT_W), jnp.float32),
        ],
    )(srccnt4, dstcnt4, ones_h, zkg16, z2_16)


def _tc_expmap0(ego):
    def body(x_ref, o_ref):
        x = x_ref[...]
        n = _norm(x)
        o_ref[...] = _proj(jnp.tanh(n) * x / n)

    return pl.pallas_call(
        body,
        grid=(25,),
        in_specs=[pl.BlockSpec((2000, DIM), lambda i: (i, 0))],
        out_specs=pl.BlockSpec((2000, DIM), lambda i: (i, 0)),
        out_shape=_sds((N_ENT, DIM)),
    )(ego)


def _tc_relsel(relm, rtx):
    def body(rm_ref, rt_ref, o_ref):
        rm = rm_ref[...]
        oh = (lax.broadcasted_iota(jnp.int32, (16, EBLK), 0)
              == (rm + 2)).astype(jnp.float32)
        o_ref[...] = jax.lax.dot_general(
            rt_ref[...], oh, (((1,), (0,)), ((), ())),
            preferred_element_type=jnp.float32,
            precision=jax.lax.Precision.HIGHEST)

    return pl.pallas_call(
        body,
        grid=(N_EBLK,),
        in_specs=[
            pl.BlockSpec((1, EBLK), lambda i: (0, i)),
            pl.BlockSpec((72, 16), lambda i: (0, 0)),
        ],
        out_specs=pl.BlockSpec((72, EBLK), lambda i: (0, i)),
        out_shape=_sds((72, EKG_PAD)),
    )(relm, rtx)


def _tc_edge_math(pT, xdT, relx):
    def body(p_ref, x_ref, r_ref, rn_ref, o_ref):
        p = p_ref[...].T
        xd = x_ref[...].T
        rl = r_ref[...]
        mx = 1.0 - 1e-3

        def rsum(v):
            return jnp.sum(v, axis=0, keepdims=True)

        P2 = rsum(p * p)
        X2 = rsum(xd * xd)
        R2 = rn_ref[0:1, :]
        PX = rsum(p * xd)
        PR = rsum(p * rl)
        XR = rsum(xd * rl)

        lam = 2.0 / jnp.maximum(1.0 - P2, 1e-10)

        def expm_coeffs(U2, PU):
            n_u = jnp.sqrt(U2 + 1e-15)
            t = jnp.tanh(lam * n_u / 2.0)
            sc = t / n_u
            s2 = sc * sc * U2
            dps = sc * PU
            a = 1.0 + 2.0 * dps + s2
            b = 1.0 - P2
            iden = 1.0 / jnp.maximum(1.0 + 2.0 * dps + P2 * s2, 1e-10)
            s_m = jnp.maximum(
                iden * iden * (a * a * P2 + 2.0 * a * b * dps + b * b * s2),
                0.0)
            n_m = jnp.sqrt(s_m + 1e-15)
            f = jnp.where(n_m > mx, mx / n_m, 1.0)
            return f * iden * a, f * iden * b * sc, f * f * s_m

        Ap, Ax, SA = expm_coeffs(X2, PX)
        Bp, Br, SB = expm_coeffs(R2, PR)

        dab = (Ap * Bp * P2 + Ap * Br * PR + Ax * Bp * PX + Ax * Br * XR)
        a1 = 1.0 + 2.0 * dab + SB
        b1 = 1.0 - SA
        id1 = 1.0 / jnp.maximum(1.0 + 2.0 * dab + SA * SB, 1e-10)
        s_m1 = jnp.maximum(
            id1 * id1 * (a1 * a1 * SA + 2.0 * a1 * b1 * dab + b1 * b1 * SB),
            0.0)
        n_m1 = jnp.sqrt(s_m1 + 1e-15)
        f1 = jnp.where(n_m1 > mx, mx / n_m1, 1.0)
        g1 = f1 * id1
        Yp = g1 * (a1 * Ap + b1 * Bp)
        Yx = g1 * a1 * Ax
        Yr = g1 * b1 * Br
        SY = f1 * f1 * s_m1

        dpy = Yp * P2 + Yx * PX + Yr * PR
        a2 = 1.0 - 2.0 * dpy + SY
        b2 = 1.0 - P2
        id2 = 1.0 / jnp.maximum(1.0 - 2.0 * dpy + P2 * SY, 1e-10)
        s_sub = jnp.maximum(
            id2 * id2 * (a2 * a2 * P2 - 2.0 * a2 * b2 * dpy + b2 * b2 * SY),
            0.0)
        n_sub = jnp.sqrt(s_sub + 1e-15)

        cn = jnp.clip(n_sub, -1.0 + EPS, 1.0 - EPS)
        at = 0.5 * jnp.log((1.0 + cn) / (1.0 - cn))
        k = jnp.maximum(1.0 - P2, 1e-10) * at / n_sub
        kid2 = k * id2
        Cp = kid2 * (b2 * Yp - a2)
        Cx = kid2 * b2 * Yx
        Cr = kid2 * b2 * Yr

        ts = (Cp * p + Cx * xd + Cr * rl).T
        o_ref[0] = ts[:, :HALF]
        o_ref[1] = ts[:, HALF:]

    return pl.pallas_call(
        body,
        grid=(N_EBLK,),
        in_specs=[
            pl.BlockSpec((EBLK, DIM), lambda i: (i, 0)),
            pl.BlockSpec((EBLK, DIM), lambda i: (i, 0)),
            pl.BlockSpec((DIM, EBLK), lambda i: (0, i)),
            pl.BlockSpec((8, EBLK), lambda i: (8, i)),
        ],
        out_specs=pl.BlockSpec((NC, EBLK, HALF), lambda i: (0, i, 0)),
        out_shape=_sds((NC, EKG_PAD, HALF)),
    )(pT, xdT, relx, relx)


def _tc_node(ego_items, item_cf, u_embed, g1, g2):
    nb_i = N_ITEMS // 1000
    nb = N_NODE // 1000

    def body(e_ref, cf_ref, u_ref, g1_ref, g2_ref, lo_ref, hi_ref):
        i = pl.program_id(0)

        @pl.when(i < nb_i)
        def _():
            e = e_ref[...]
            icf = cf_ref[...]
            dn = (((1,), (1,)), ((), ()))
            z = (lax.dot_general(e, g1_ref[...], dn,
                                 preferred_element_type=jnp.float32,
                                 precision=jax.lax.Precision.HIGHEST)
                 + lax.dot_general(icf, g2_ref[...], dn,
                                   preferred_element_type=jnp.float32,
                                   precision=jax.lax.Precision.HIGHEST))
            gi = jax.nn.sigmoid(z)
            fus = gi * e + (1.0 - gi) * icf
            lo_ref[...] = fus[:, :HALF]
            hi_ref[...] = fus[:, HALF:]

        @pl.when(i >= nb_i)
        def _():
            u = u_ref[...]
            lo_ref[...] = u[:, :HALF]
            hi_ref[...] = u[:, HALF:]

    return pl.pallas_call(
        body,
        grid=(nb,),
        in_specs=[
            pl.BlockSpec((1000, DIM), lambda i: (jnp.minimum(i, nb_i - 1), 0)),
            pl.BlockSpec((1000, DIM), lambda i: (jnp.minimum(i, nb_i - 1), 0)),
            pl.BlockSpec((1000, DIM), lambda i: (jnp.maximum(i - nb_i, 0), 0)),
            pl.BlockSpec((DIM, DIM), lambda i: (0, 0)),
            pl.BlockSpec((DIM, DIM), lambda i: (0, 0)),
        ],
        out_specs=[
            pl.BlockSpec((1000, HALF), lambda i: (i, 0)),
            pl.BlockSpec((1000, HALF), lambda i: (i, 0)),
        ],
        out_shape=(_sds((N_NODE, HALF)), _sds((N_NODE, HALF))),
    )(ego_items, item_cf, u_embed, g1, g2)


def _finalize_body(s2_ref, c2_ref, res_ref, new_ref, ro_ref):
    s = jnp.concatenate([s2_ref[0], s2_ref[1]], axis=1)
    cnt = c2_ref[0][:, :1] + c2_ref[1][:, :1]
    m = s / jnp.maximum(cnt, 1.0)
    e = _l2n(m)
    new_ref[...] = e
    ro_ref[...] = res_ref[...] + e


def _tc_fin(s2, c2, res, row_off, n_rows):
    nb = n_rows // 1000

    return pl.pallas_call(
        _finalize_body,
        grid=(nb,),
        in_specs=[
            pl.BlockSpec((NC, 1000, HALF), lambda i: (0, i + row_off, 0)),
            pl.BlockSpec((NC, 1000, CNT_W), lambda i: (0, i + row_off, 0)),
            pl.BlockSpec((1000, DIM), lambda i: (i, 0)),
        ],
        out_specs=[
            pl.BlockSpec((1000, DIM), lambda i: (i, 0)),
            pl.BlockSpec((1000, DIM), lambda i: (i, 0)),
        ],
        out_shape=(_sds((n_rows, DIM)), _sds((n_rows, DIM))),
    )(s2, c2, res)


def kernel(kg_edge_index, kg_edge_type, i2u_edge_index, entity_user_embed,
           relation_embed, items_embed_cf, gate1_w, gate2_w):
    src = kg_edge_index[0].astype(jnp.int32)
    dst = kg_edge_index[1].astype(jnp.int32)
    i2s = i2u_edge_index[0].astype(jnp.int32)
    i2d = i2u_edge_index[1].astype(jnp.int32)
    ktype = kg_edge_type.astype(jnp.int32)

    pad_kg = EKG_PAD - E_KG
    pad_2 = E2_PAD - E_I2U
    srcg4 = jnp.pad(src, (0, pad_kg)).reshape(NW, KG_CHG, CH)
    dstg4 = jnp.pad(dst, (0, pad_kg)).reshape(NW, KG_CHG, CH)
    src_s = jnp.pad(src, (0, pad_kg), constant_values=N_ENT)
    srcs4 = src_s.reshape(NS, KG_CHS // KI, KI, CH)
    srccnt4 = src_s.reshape(NW, KG_CHG // KI, KI, CH)
    s2g = jnp.pad(i2s, (0, pad_2))
    src5 = jnp.stack([s2g, s2g + N_NODE]).reshape(
        NC, NS, I2U_CHS // KI, KI, CH)
    d2_s = jnp.pad(i2d, (0, pad_2), constant_values=N_NODE)
    dst4 = d2_s.reshape(NS, I2U_CHS // KI, KI, CH)
    dstcnt4 = d2_s.reshape(NW, I2U_CHG // KI, KI, CH)
    relm = jnp.pad(ktype, (0, pad_kg)).reshape(1, EKG_PAD)
    rtx = jnp.zeros((72, 16), jnp.float32)
    rtx = rtx.at[:DIM, :N_REL].set(relation_embed.T)
    rtx = rtx.at[DIM, :N_REL].set(jnp.sum(relation_embed ** 2, axis=1))

    zeros_kg = jnp.zeros((ACC_KG, HALF), jnp.float32)
    zeros_2 = jnp.zeros((ACC_2, HALF), jnp.float32)
    zkg16 = jnp.zeros((ACC_KG, CNT_W), jnp.float32)
    z2_16 = jnp.zeros((ACC_2, CNT_W), jnp.float32)
    ones16 = jnp.ones((CH, CNT_W), jnp.float32)

    ckg, c2 = _sc_counts(srccnt4, dstcnt4, ones16, zkg16, z2_16)
    relx = _tc_relsel(relm, rtx)

    ego = entity_user_embed[:N_ENT]
    u_embed = entity_user_embed[N_ENT:]
    item_cf = items_embed_cf
    ent_res, user_res, item_res = ego, u_embed, item_cf

    for layer in range(2):
        P = _tc_expmap0(ego)
        n_lo, n_hi = _tc_node(ego[:N_ITEMS], item_cf, u_embed,
                              gate1_w[layer], gate2_w[layer])
        node2 = jnp.concatenate([n_lo, n_hi], axis=0)
        p_src, x_dst = _sc_gather_kg(P, ego, srcg4, dstg4)
        a2 = _sc_i2u(node2, src5, dst4, zeros_2)
        ts2 = _tc_edge_math(p_src, x_dst, relx)
        kg2 = _sc_scatter_kg(ts2, srcs4, zeros_kg)
        ego, ent_res = _tc_fin(kg2, ckg, ent_res, 0, N_ENT)
        item_cf, item_res = _tc_fin(a2, c2, item_res, 0, N_ITEMS)
        u_embed, user_res = _tc_fin(a2, c2, user_res, N_ITEMS // 1000,
                                    N_USERS)

    return ent_res, user_res, item_res


# speedup vs baseline: 2.3460x; 1.8069x over previous
import jax
import jax.numpy as jnp
from jax import lax
from jax.experimental import pallas as pl
from jax.experimental.pallas import tpu as pltpu
from jax.experimental.pallas import tpu_sc as plsc

N_ENT = 50000
N_USERS = 10000
N_ITEMS = 20000
N_REL = 10
DIM = 64
HALF = 32
E_KG = 800000
E_I2U = 600000
EPS = 1e-7

NC = 2
NS = 16
NW = NC * NS
CH = 128

KG_CHG = 196
TILE_KG_G = KG_CHG * CH
EKG_PAD = NW * TILE_KG_G
KG_CHS = 2 * KG_CHG
TILE_KG_S = KG_CHS * CH

I2U_CHG = 147
TILE_I2U_G = I2U_CHG * CH
E2_PAD = NW * TILE_I2U_G
I2U_CHS = 2 * I2U_CHG
TILE_I2U_S = I2U_CHS * CH

ACC_KG = 50048
ZKG = ACC_KG // NS
N_NODE = N_ITEMS + N_USERS
ACC_2 = 30080
Z2 = ACC_2 // NS
CNT_W = 16

EBLK = 2048
N_EBLK = EKG_PAD // EBLK

KI = 7

_SC_MESH = dict(core_axis_name="c", subcore_axis_name="s")


def _sds(shape, dtype=jnp.float32):
    return jax.ShapeDtypeStruct(shape, dtype)


def _norm(x):
    return jnp.sqrt(jnp.sum(x * x, axis=-1, keepdims=True) + 1e-15)


def _proj(x):
    maxnorm = 1.0 - 1e-3
    n = _norm(x)
    return jnp.where(n > maxnorm, x / n * maxnorm, x)


def _l2n(x):
    return x / _norm(x)


def _sc_gather_kg(P, ego, srcg4, dstg4):
    def body(p_h, ego_h, src_h, dst_h, px_out,
             idx_s, idx_d, pa, ea, pb, eb,
             gsa, gsb, wsa, wsb):
        c = lax.axis_index("c")
        s = lax.axis_index("s")
        wid = s * NC + c
        base = wid * TILE_KG_G
        pltpu.sync_copy(src_h.at[wid], idx_s)
        pltpu.sync_copy(dst_h.at[wid], idx_d)

        def fire_g(j, rp, re, sem):
            pltpu.async_copy(p_h.at[idx_s.at[j]], rp, sem)
            pltpu.async_copy(ego_h.at[idx_d.at[j]], re, sem)

        def wait_g(rp, re, sem):
            pltpu.make_async_copy(p_h.at[pl.ds(0, CH)], rp, sem).wait()
            pltpu.make_async_copy(p_h.at[pl.ds(0, CH)], re, sem).wait()

        def fire_w(j, rp, re, sem):
            off = base + j * CH
            pltpu.async_copy(rp, px_out.at[pl.ds(off, CH), pl.ds(0, DIM)],
                             sem)
            pltpu.async_copy(re, px_out.at[pl.ds(off, CH), pl.ds(DIM, DIM)],
                             sem)

        def wait_w(rp, re, sem):
            pltpu.make_async_copy(
                rp, px_out.at[pl.ds(0, CH), pl.ds(0, DIM)], sem).wait()
            pltpu.make_async_copy(
                re, px_out.at[pl.ds(0, CH), pl.ds(DIM, DIM)], sem).wait()

        fire_g(0, pa, ea, gsa)
        fire_g(1, pb, eb, gsb)
        n_it = KG_CHG // 2

        def step(o, carry):
            a = 2 * o
            wait_g(pa, ea, gsa)
            fire_w(a, pa, ea, wsa)
            wait_g(pb, eb, gsb)
            fire_w(a + 1, pb, eb, wsb)
            wait_w(pa, ea, wsa)

            @pl.when(o < n_it - 1)
            def _():
                fire_g(a + 2, pa, ea, gsa)

            wait_w(pb, eb, wsb)

            @pl.when(o < n_it - 1)
            def _():
                fire_g(a + 3, pb, eb, gsb)

            return carry

        lax.fori_loop(0, n_it, step, 0)

    return pl.kernel(
        body,
        out_type=_sds((EKG_PAD, 2 * DIM)),
        mesh=plsc.VectorSubcoreMesh(**_SC_MESH),
        compiler_params=pltpu.CompilerParams(use_tc_tiling_on_sc=False),
        scratch_types=[
            pltpu.VMEM((KG_CHG, CH), jnp.int32),
            pltpu.VMEM((KG_CHG, CH), jnp.int32),
            pltpu.VMEM((CH, DIM), jnp.float32),
            pltpu.VMEM((CH, DIM), jnp.float32),
            pltpu.VMEM((CH, DIM), jnp.float32),
            pltpu.VMEM((CH, DIM), jnp.float32),
            pltpu.SemaphoreType.DMA,
            pltpu.SemaphoreType.DMA,
            pltpu.SemaphoreType.DMA,
            pltpu.SemaphoreType.DMA,
        ],
    )(P, ego, srcg4, dstg4)


def _sc_scatter_kg(ts2, srcs4, zeros_kg):
    def body(ts_h, src_h, z_h, out_h, idx0, idx1, r0, r1, acc,
             is0, is1, rs0, rs1):
        c = lax.axis_index("c")
        s = lax.axis_index("s")
        pltpu.sync_copy(z_h.at[pl.ds(s * ZKG, ZKG)],
                        acc.at[pl.ds(s * ZKG, ZKG)])
        plsc.subcore_barrier()
        base = s * TILE_KG_S
        rbuf = (r0, r1)
        rsem = (rs0, rs1)

        def proc_group(g, idx_v):
            off0 = base + g * KI * CH
            pltpu.async_copy(
                ts_h.at[pl.ds(off0, CH), pl.ds(c * HALF, HALF)], r0, rs0)
            for kk in range(KI):
                cur, csem = rbuf[kk % 2], rsem[kk % 2]
                if kk < KI - 1:
                    off = base + (g * KI + kk + 1) * CH
                    pltpu.async_copy(
                        ts_h.at[pl.ds(off, CH), pl.ds(c * HALF, HALF)],
                        rbuf[(kk + 1) % 2], rsem[(kk + 1) % 2])
                pltpu.make_async_copy(
                    ts_h.at[pl.ds(0, CH), pl.ds(0, HALF)], cur, csem).wait()
                pltpu.sync_copy(cur, acc.at[idx_v.at[kk]], add=True)

        n_pair = KG_CHS // KI // 2
        pltpu.async_copy(src_h.at[s, 0], idx0, is0)

        def step(m, carry):
            pltpu.make_async_copy(src_h.at[s, 0], idx0, is0).wait()
            pltpu.async_copy(src_h.at[s, 2 * m + 1], idx1, is1)
            proc_group(2 * m, idx0)
            pltpu.make_async_copy(src_h.at[s, 0], idx1, is1).wait()

            @pl.when(m < n_pair - 1)
            def _():
                pltpu.async_copy(src_h.at[s, 2 * m + 2], idx0, is0)

            proc_group(2 * m + 1, idx1)
            return carry

        lax.fori_loop(0, n_pair, step, 0)
        plsc.subcore_barrier()
        pltpu.sync_copy(acc.at[pl.ds(s * ZKG, ZKG)],
                        out_h.at[c, pl.ds(s * ZKG, ZKG)])

    return pl.kernel(
        body,
        out_type=_sds((NC, ACC_KG, HALF)),
        mesh=plsc.VectorSubcoreMesh(**_SC_MESH),
        compiler_params=pltpu.CompilerParams(use_tc_tiling_on_sc=False),
        scratch_types=[
            pltpu.VMEM((KI, CH), jnp.int32),
            pltpu.VMEM((KI, CH), jnp.int32),
            pltpu.VMEM((CH, HALF), jnp.float32),
            pltpu.VMEM((CH, HALF), jnp.float32),
            pltpu.VMEM_SHARED((ACC_KG, HALF), jnp.float32),
            pltpu.SemaphoreType.DMA,
            pltpu.SemaphoreType.DMA,
            pltpu.SemaphoreType.DMA,
            pltpu.SemaphoreType.DMA,
        ],
    )(ts2, srcs4, zeros_kg)


def _sc_i2u(node2, src5, dst4, zeros_2):
    def body(n_h, src_h, dst_h, z_h, out_h, is0v, is1v, id0v, id1v,
             r0, r1, acc, is0, is1, rs0, rs1):
        c = lax.axis_index("c")
        s = lax.axis_index("s")
        pltpu.sync_copy(z_h.at[pl.ds(s * Z2, Z2)], acc.at[pl.ds(s * Z2, Z2)])
        plsc.subcore_barrier()
        rbuf = (r0, r1)
        rsem = (rs0, rs1)

        def proc_group(sv, dv):
            pltpu.async_copy(n_h.at[sv.at[0]], r0, rs0)
            for kk in range(KI):
                cur, csem = rbuf[kk % 2], rsem[kk % 2]
                if kk < KI - 1:
                    pltpu.async_copy(n_h.at[sv.at[kk + 1]],
                                     rbuf[(kk + 1) % 2], rsem[(kk + 1) % 2])
                pltpu.make_async_copy(n_h.at[pl.ds(0, CH)], cur, csem).wait()
                pltpu.sync_copy(cur, acc.at[dv.at[kk]], add=True)

        n_pair = I2U_CHS // KI // 2
        pltpu.async_copy(src_h.at[c, s, 0], is0v, is0)
        pltpu.async_copy(dst_h.at[s, 0], id0v, is0)

        def step(m, carry):
            pltpu.make_async_copy(src_h.at[c, s, 0], is0v, is0).wait()
            pltpu.make_async_copy(dst_h.at[s, 0], id0v, is0).wait()
            pltpu.async_copy(src_h.at[c, s, 2 * m + 1], is1v, is1)
            pltpu.async_copy(dst_h.at[s, 2 * m + 1], id1v, is1)
            proc_group(is0v, id0v)
            pltpu.make_async_copy(src_h.at[c, s, 0], is1v, is1).wait()
            pltpu.make_async_copy(dst_h.at[s, 0], id1v, is1).wait()

            @pl.when(m < n_pair - 1)
            def _():
                pltpu.async_copy(src_h.at[c, s, 2 * m + 2], is0v, is0)
                pltpu.async_copy(dst_h.at[s, 2 * m + 2], id0v, is0)

            proc_group(is1v, id1v)
            return carry

        lax.fori_loop(0, n_pair, step, 0)
        plsc.subcore_barrier()
        pltpu.sync_copy(acc.at[pl.ds(s * Z2, Z2)],
                        out_h.at[c, pl.ds(s * Z2, Z2)])

    return pl.kernel(
        body,
        out_type=_sds((NC, ACC_2, HALF)),
        mesh=plsc.VectorSubcoreMesh(**_SC_MESH),
        compiler_params=pltpu.CompilerParams(use_tc_tiling_on_sc=False),
        scratch_types=[
            pltpu.VMEM((KI, CH), jnp.int32),
            pltpu.VMEM((KI, CH), jnp.int32),
            pltpu.VMEM((KI, CH), jnp.int32),
            pltpu.VMEM((KI, CH), jnp.int32),
            pltpu.VMEM((CH, HALF), jnp.float32),
            pltpu.VMEM((CH, HALF), jnp.float32),
            pltpu.VMEM_SHARED((ACC_2, HALF), jnp.float32),
            pltpu.SemaphoreType.DMA,
            pltpu.SemaphoreType.DMA,
            pltpu.SemaphoreType.DMA,
            pltpu.SemaphoreType.DMA,
        ],
    )(node2, src5, dst4, zeros_2)


def _sc_counts(srccnt4, dstcnt4, ones_h, zkg16, z2_16):
    def body(src_h, dst_h, on_h, zk_h, z2_h, okg, o2,
             idx_kg, idx_2, ones_v, acck, acc2):
        c = lax.axis_index("c")
        s = lax.axis_index("s")
        wid = s * NC + c
        pltpu.sync_copy(zk_h.at[pl.ds(s * ZKG, ZKG)],
                        acck.at[pl.ds(s * ZKG, ZKG)])
        pltpu.sync_copy(z2_h.at[pl.ds(s * Z2, Z2)],
                        acc2.at[pl.ds(s * Z2, Z2)])
        pltpu.sync_copy(on_h, ones_v)
        plsc.subcore_barrier()

        def chunk_kg(o, carry):
            pltpu.sync_copy(src_h.at[wid, o], idx_kg)
            for kk in range(KI):
                pltpu.sync_copy(ones_v, acck.at[idx_kg.at[kk]], add=True)
            return carry

        lax.fori_loop(0, KG_CHG // KI, chunk_kg, 0)

        def chunk_2(o, carry):
            pltpu.sync_copy(dst_h.at[wid, o], idx_2)
            for kk in range(KI):
                pltpu.sync_copy(ones_v, acc2.at[idx_2.at[kk]], add=True)
            return carry

        lax.fori_loop(0, I2U_CHG // KI, chunk_2, 0)
        plsc.subcore_barrier()
        pltpu.sync_copy(acck.at[pl.ds(s * ZKG, ZKG)],
                        okg.at[c, pl.ds(s * ZKG, ZKG)])
        pltpu.sync_copy(acc2.at[pl.ds(s * Z2, Z2)],
                        o2.at[c, pl.ds(s * Z2, Z2)])

    return pl.kernel(
        body,
        out_type=(_sds((NC, ACC_KG, CNT_W)), _sds((NC, ACC_2, CNT_W))),
        mesh=plsc.VectorSubcoreMesh(**_SC_MESH),
        compiler_params=pltpu.CompilerParams(use_tc_tiling_on_sc=False),
        scratch_types=[
            pltpu.VMEM((KI, CH), jnp.int32),
            pltpu.VMEM((KI, CH), jnp.int32),
            pltpu.VMEM((CH, CNT_W), jnp.float32),
            pltpu.VMEM_SHARED((ACC_KG, CNT_W), jnp.float32),
            pltpu.VMEM_SHARED((ACC_2, CNT_W), jnp.float32),
        ],
    )(srccnt4, dstcnt4, ones_h, zkg16, z2_16)


def _tc_expmap0(ego):
    def body(x_ref, o_ref):
        x = x_ref[...]
        n = _norm(x)
        o_ref[...] = _proj(jnp.tanh(n) * x / n)

    return pl.pallas_call(
        body,
        grid=(25,),
        in_specs=[pl.BlockSpec((2000, DIM), lambda i: (i, 0))],
        out_specs=pl.BlockSpec((2000, DIM), lambda i: (i, 0)),
        out_shape=_sds((N_ENT, DIM)),
    )(ego)


def _tc_relsel(relm, rtx):
    def body(rm_ref, rt_ref, o_ref):
        rm = rm_ref[...]
        oh = (lax.broadcasted_iota(jnp.int32, (16, EBLK), 0)
              == (rm + 2)).astype(jnp.float32)
        o_ref[...] = jax.lax.dot_general(
            rt_ref[...], oh, (((1,), (0,)), ((), ())),
            preferred_element_type=jnp.float32,
            precision=jax.lax.Precision.HIGHEST)

    return pl.pallas_call(
        body,
        grid=(N_EBLK,),
        in_specs=[
            pl.BlockSpec((1, EBLK), lambda i: (0, i)),
            pl.BlockSpec((72, 16), lambda i: (0, 0)),
        ],
        out_specs=pl.BlockSpec((72, EBLK), lambda i: (0, i)),
        out_shape=_sds((72, EKG_PAD)),
    )(relm, rtx)


def _tc_edge_math(px, relx):
    def body(px_ref, r_ref, rn_ref, o_ref):
        both = px_ref[...].T
        p = both[:DIM]
        xd = both[DIM:]
        rl = r_ref[...]
        mx = 1.0 - 1e-3

        def rsum(v):
            return jnp.sum(v, axis=0, keepdims=True)

        P2 = rsum(p * p)
        X2 = rsum(xd * xd)
        R2 = rn_ref[0:1, :]
        PX = rsum(p * xd)
        PR = rsum(p * rl)
        XR = rsum(xd * rl)

        lam = 2.0 / jnp.maximum(1.0 - P2, 1e-10)

        def expm_coeffs(U2, PU):
            n_u = jnp.sqrt(U2 + 1e-15)
            t = jnp.tanh(lam * n_u / 2.0)
            sc = t / n_u
            s2 = sc * sc * U2
            dps = sc * PU
            a = 1.0 + 2.0 * dps + s2
            b = 1.0 - P2
            iden = 1.0 / jnp.maximum(1.0 + 2.0 * dps + P2 * s2, 1e-10)
            s_m = jnp.maximum(
                iden * iden * (a * a * P2 + 2.0 * a * b * dps + b * b * s2),
                0.0)
            n_m = jnp.sqrt(s_m + 1e-15)
            f = jnp.where(n_m > mx, mx / n_m, 1.0)
            return f * iden * a, f * iden * b * sc, f * f * s_m

        Ap, Ax, SA = expm_coeffs(X2, PX)
        Bp, Br, SB = expm_coeffs(R2, PR)

        dab = (Ap * Bp * P2 + Ap * Br * PR + Ax * Bp * PX + Ax * Br * XR)
        a1 = 1.0 + 2.0 * dab + SB
        b1 = 1.0 - SA
        id1 = 1.0 / jnp.maximum(1.0 + 2.0 * dab + SA * SB, 1e-10)
        s_m1 = jnp.maximum(
            id1 * id1 * (a1 * a1 * SA + 2.0 * a1 * b1 * dab + b1 * b1 * SB),
            0.0)
        n_m1 = jnp.sqrt(s_m1 + 1e-15)
        f1 = jnp.where(n_m1 > mx, mx / n_m1, 1.0)
        g1 = f1 * id1
        Yp = g1 * (a1 * Ap + b1 * Bp)
        Yx = g1 * a1 * Ax
        Yr = g1 * b1 * Br
        SY = f1 * f1 * s_m1

        dpy = Yp * P2 + Yx * PX + Yr * PR
        a2 = 1.0 - 2.0 * dpy + SY
        b2 = 1.0 - P2
        id2 = 1.0 / jnp.maximum(1.0 - 2.0 * dpy + P2 * SY, 1e-10)
        s_sub = jnp.maximum(
            id2 * id2 * (a2 * a2 * P2 - 2.0 * a2 * b2 * dpy + b2 * b2 * SY),
            0.0)
        n_sub = jnp.sqrt(s_sub + 1e-15)

        cn = jnp.clip(n_sub, -1.0 + EPS, 1.0 - EPS)
        at = 0.5 * jnp.log((1.0 + cn) / (1.0 - cn))
        k = jnp.maximum(1.0 - P2, 1e-10) * at / n_sub
        kid2 = k * id2
        Cp = kid2 * (b2 * Yp - a2)
        Cx = kid2 * b2 * Yx
        Cr = kid2 * b2 * Yr

        ts = (Cp * p + Cx * xd + Cr * rl).T
        o_ref[...] = jnp.concatenate(
            [ts, jnp.zeros((EBLK, DIM), jnp.float32)], axis=1)

    return pl.pallas_call(
        body,
        grid=(N_EBLK,),
        in_specs=[
            pl.BlockSpec((EBLK, 2 * DIM), lambda i: (i, 0)),
            pl.BlockSpec((DIM, EBLK), lambda i: (0, i)),
            pl.BlockSpec((8, EBLK), lambda i: (8, i)),
        ],
        out_specs=pl.BlockSpec((EBLK, 2 * DIM), lambda i: (i, 0)),
        out_shape=_sds((EKG_PAD, 2 * DIM)),
    )(px, relx, relx)


def _tc_node(ego_items, item_cf, u_embed, g1, g2):
    nb_i = N_ITEMS // 1000
    nb = N_NODE // 1000

    def body(e_ref, cf_ref, u_ref, g1_ref, g2_ref, lo_ref, hi_ref):
        i = pl.program_id(0)

        @pl.when(i < nb_i)
        def _():
            e = e_ref[...]
            icf = cf_ref[...]
            dn = (((1,), (1,)), ((), ()))
            z = (lax.dot_general(e, g1_ref[...], dn,
                                 preferred_element_type=jnp.float32,
                                 precision=jax.lax.Precision.HIGHEST)
                 + lax.dot_general(icf, g2_ref[...], dn,
                                   preferred_element_type=jnp.float32,
                                   precision=jax.lax.Precision.HIGHEST))
            gi = jax.nn.sigmoid(z)
            fus = gi * e + (1.0 - gi) * icf
            lo_ref[...] = fus[:, :HALF]
            hi_ref[...] = fus[:, HALF:]

        @pl.when(i >= nb_i)
        def _():
            u = u_ref[...]
            lo_ref[...] = u[:, :HALF]
            hi_ref[...] = u[:, HALF:]

    return pl.pallas_call(
        body,
        grid=(nb,),
        in_specs=[
            pl.BlockSpec((1000, DIM), lambda i: (jnp.minimum(i, nb_i - 1), 0)),
            pl.BlockSpec((1000, DIM), lambda i: (jnp.minimum(i, nb_i - 1), 0)),
            pl.BlockSpec((1000, DIM), lambda i: (jnp.maximum(i - nb_i, 0), 0)),
            pl.BlockSpec((DIM, DIM), lambda i: (0, 0)),
            pl.BlockSpec((DIM, DIM), lambda i: (0, 0)),
        ],
        out_specs=[
            pl.BlockSpec((1000, HALF), lambda i: (i, 0)),
            pl.BlockSpec((1000, HALF), lambda i: (i, 0)),
        ],
        out_shape=(_sds((N_NODE, HALF)), _sds((N_NODE, HALF))),
    )(ego_items, item_cf, u_embed, g1, g2)


def _finalize_body(s2_ref, c2_ref, res_ref, new_ref, ro_ref):
    s = jnp.concatenate([s2_ref[0], s2_ref[1]], axis=1)
    cnt = c2_ref[0][:, :1] + c2_ref[1][:, :1]
    m = s / jnp.maximum(cnt, 1.0)
    e = _l2n(m)
    new_ref[...] = e
    ro_ref[...] = res_ref[...] + e


def _tc_fin(s2, c2, res, row_off, n_rows):
    nb = n_rows // 1000

    return pl.pallas_call(
        _finalize_body,
        grid=(nb,),
        in_specs=[
            pl.BlockSpec((NC, 1000, HALF), lambda i: (0, i + row_off, 0)),
            pl.BlockSpec((NC, 1000, CNT_W), lambda i: (0, i + row_off, 0)),
            pl.BlockSpec((1000, DIM), lambda i: (i, 0)),
        ],
        out_specs=[
            pl.BlockSpec((1000, DIM), lambda i: (i, 0)),
            pl.BlockSpec((1000, DIM), lambda i: (i, 0)),
        ],
        out_shape=(_sds((n_rows, DIM)), _sds((n_rows, DIM))),
    )(s2, c2, res)


def kernel(kg_edge_index, kg_edge_type, i2u_edge_index, entity_user_embed,
           relation_embed, items_embed_cf, gate1_w, gate2_w):
    src = kg_edge_index[0].astype(jnp.int32)
    dst = kg_edge_index[1].astype(jnp.int32)
    i2s = i2u_edge_index[0].astype(jnp.int32)
    i2d = i2u_edge_index[1].astype(jnp.int32)
    ktype = kg_edge_type.astype(jnp.int32)

    pad_kg = EKG_PAD - E_KG
    pad_2 = E2_PAD - E_I2U
    srcg4 = jnp.pad(src, (0, pad_kg)).reshape(NW, KG_CHG, CH)
    dstg4 = jnp.pad(dst, (0, pad_kg)).reshape(NW, KG_CHG, CH)
    src_s = jnp.pad(src, (0, pad_kg), constant_values=N_ENT)
    srcs4 = src_s.reshape(NS, KG_CHS // KI, KI, CH)
    srccnt4 = src_s.reshape(NW, KG_CHG // KI, KI, CH)
    s2g = jnp.pad(i2s, (0, pad_2))
    src5 = jnp.stack([s2g, s2g + N_NODE]).reshape(
        NC, NS, I2U_CHS // KI, KI, CH)
    d2_s = jnp.pad(i2d, (0, pad_2), constant_values=N_NODE)
    dst4 = d2_s.reshape(NS, I2U_CHS // KI, KI, CH)
    dstcnt4 = d2_s.reshape(NW, I2U_CHG // KI, KI, CH)
    relm = jnp.pad(ktype, (0, pad_kg)).reshape(1, EKG_PAD)
    rtx = jnp.zeros((72, 16), jnp.float32)
    rtx = rtx.at[:DIM, :N_REL].set(relation_embed.T)
    rtx = rtx.at[DIM, :N_REL].set(jnp.sum(relation_embed ** 2, axis=1))

    zeros_kg = jnp.zeros((ACC_KG, HALF), jnp.float32)
    zeros_2 = jnp.zeros((ACC_2, HALF), jnp.float32)
    zkg16 = jnp.zeros((ACC_KG, CNT_W), jnp.float32)
    z2_16 = jnp.zeros((ACC_2, CNT_W), jnp.float32)
    ones16 = jnp.ones((CH, CNT_W), jnp.float32)

    ckg, c2 = _sc_counts(srccnt4, dstcnt4, ones16, zkg16, z2_16)
    relx = _tc_relsel(relm, rtx)

    ego = entity_user_embed[:N_ENT]
    u_embed = entity_user_embed[N_ENT:]
    item_cf = items_embed_cf
    ent_res, user_res, item_res = ego, u_embed, item_cf

    for layer in range(2):
        P = _tc_expmap0(ego)
        n_lo, n_hi = _tc_node(ego[:N_ITEMS], item_cf, u_embed,
                              gate1_w[layer], gate2_w[layer])
        node2 = jnp.concatenate([n_lo, n_hi], axis=0)
        px = _sc_gather_kg(P, ego, srcg4, dstg4)
        a2 = _sc_i2u(node2, src5, dst4, zeros_2)
        ts2 = _tc_edge_math(px, relx)
        kg2 = _sc_scatter_kg(ts2, srcs4, zeros_kg)
        ego, ent_res = _tc_fin(kg2, ckg, ent_res, 0, N_ENT)
        item_cf, item_res = _tc_fin(a2, c2, item_res, 0, N_ITEMS)
        u_embed, user_res = _tc_fin(a2, c2, user_res, N_ITEMS // 1000,
                                    N_USERS)

    return ent_res, user_res, item_res
